# baseline (speedup 1.0000x reference)
import functools

import jax
import jax.numpy as jnp
from jax import lax
from jax.experimental import pallas as pl
from jax.experimental.pallas import tpu as pltpu
from jax.experimental.pallas import tpu_sc as plsc

F32 = jnp.float32
I32 = jnp.int32
N, N1, N2 = 100000, 25000, 5000
E1, E2 = 400000, 80000
H = 64
NCLS = 20
NWORK = 32
P1 = 25600
P2 = 5120
NACC1 = P1 + 128
NACC2 = P2 + 128


def _mesh():
    return plsc.VectorSubcoreMesh(core_axis_name="c", subcore_axis_name="s")


def _wid():
    return lax.axis_index("s") * 2 + lax.axis_index("c")


def _sc_gather(table, idx2, V, Bp):
    cpw = Bp // NWORK
    nb = cpw // 512
    kpc = cpw // 128
    assert cpw % 512 == 0

    def body(tab_ref, idx_ref, out_ref, idx_v, bufs, g0, g1, w0, w1):
        w = _wid()
        pltpu.sync_copy(idx_ref.at[pl.ds(w * kpc, kpc)], idx_v)
        gsems = (g0, g1)
        wsems = (w0, w1)

        def fire(b, slot):
            for j in range(4):
                pltpu.make_async_copy(
                    tab_ref.at[idx_v.at[b * 4 + j]],
                    bufs.at[slot, pl.ds(j * 128, 128)],
                    gsems[slot]).start()

        def gwait(slot):
            pltpu.make_async_copy(
                tab_ref.at[pl.ds(0, 512)], bufs.at[slot], gsems[slot]).wait()

        def wstart(b, slot):
            pltpu.async_copy(
                bufs.at[slot], out_ref.at[pl.ds(w * cpw + b * 512, 512)],
                wsems[slot])

        def wwait(slot):
            pltpu.make_async_copy(
                bufs.at[slot], out_ref.at[pl.ds(0, 512)], wsems[slot]).wait()

        fire(0, 0)

        def it(b, slot):
            nxt = 1 - slot

            @pl.when(b >= 1)
            def _():
                wwait(nxt)

            @pl.when(b + 1 < nb)
            def _():
                fire(b + 1, nxt)

            gwait(slot)
            wstart(b, slot)

        def loop_body(b, carry):
            @pl.when(b % 2 == 0)
            def _():
                it(b, 0)

            @pl.when(b % 2 == 1)
            def _():
                it(b, 1)

            return carry

        lax.fori_loop(0, nb, loop_body, 0)
        wwait((nb - 1) % 2)

    f = functools.partial(
        pl.kernel,
        out_type=jax.ShapeDtypeStruct((Bp, H), F32),
        mesh=_mesh(),
        compiler_params=pltpu.CompilerParams(use_tc_tiling_on_sc=False, needs_layout_passes=False),
        scratch_types=[
            pltpu.VMEM((kpc, 128), I32),
            pltpu.VMEM((2, 512, H), F32),
            pltpu.SemaphoreType.DMA,
            pltpu.SemaphoreType.DMA,
            pltpu.SemaphoreType.DMA,
            pltpu.SemaphoreType.DMA,
        ],
    )(body)
    return f(table, idx2)


def _sc_ffn_add(pf, lab2, zero_acc, Np):
    cpw = Np // NWORK
    kpc = cpw // 128
    nacc = NACC1

    def body(pf_ref, lab_ref, zero_ref, out_ref, idx_v, rows_v, acc):
        w = _wid()
        s = lax.axis_index("s")
        c = lax.axis_index("c")
        stripe = nacc // 16
        pltpu.sync_copy(zero_ref.at[pl.ds(s * stripe, stripe)],
                        acc.at[pl.ds(s * stripe, stripe)])
        plsc.subcore_barrier()
        pltpu.sync_copy(lab_ref.at[pl.ds(w * kpc, kpc)], idx_v)

        def loop_body(j, carry):
            pltpu.sync_copy(pf_ref.at[pl.ds(w * cpw + j * 128, 128)], rows_v)
            pltpu.sync_copy(rows_v, acc.at[idx_v.at[j]], add=True)
            return carry

        lax.fori_loop(0, kpc, loop_body, 0)
        plsc.subcore_barrier()

        @pl.when(s == 0)
        def _():
            pltpu.sync_copy(acc.at[pl.ds(0, P1)], out_ref.at[c])

    f = functools.partial(
        pl.kernel,
        out_type=jax.ShapeDtypeStruct((2, P1, 16), F32),
        mesh=_mesh(),
        compiler_params=pltpu.CompilerParams(use_tc_tiling_on_sc=False, needs_layout_passes=False),
        scratch_types=[
            pltpu.VMEM((kpc, 128), I32),
            pltpu.VMEM((128, 16), F32),
            pltpu.VMEM_SHARED((nacc, 16), F32),
        ],
    )(body)
    return f(pf, lab2, zero_acc)


def _sc_smax(zt, dst, n_out, n_acc, Ep):
    nc = Ep // 2048
    assert Ep % 2048 == 0

    def body(zt_ref, dst_ref, out_ref, zb, db, acc_a, acc_b, f0, f1):
        w = _wid()

        def zero_body(i, carry):
            z16 = jnp.zeros((16,), F32)
            acc_a[pl.ds(i * 16, 16)] = z16
            acc_b[pl.ds(i * 16, 16)] = z16
            return carry

        lax.fori_loop(0, n_acc // 16, zero_body, 0)
        fsems = (f0, f1)

        def fetch(ci, slot):
            for f in range(2):
                pltpu.make_async_copy(
                    zt_ref.at[pl.ds((2 * w + f) * Ep + ci * 2048, 2048)],
                    zb.at[slot, f], fsems[slot]).start()
            pltpu.make_async_copy(
                dst_ref.at[pl.ds(ci * 2048, 2048)], db.at[slot],
                fsems[slot]).start()

        def fwait(slot):
            for f in range(2):
                pltpu.make_async_copy(
                    zt_ref.at[pl.ds(0, 2048)], zb.at[slot, f],
                    fsems[slot]).wait()
            pltpu.make_async_copy(
                dst_ref.at[pl.ds(0, 2048)], db.at[slot], fsems[slot]).wait()

        def process(slot):
            def group(g, carry):
                dv = db[slot, pl.ds(g * 16, 16)]
                z0 = zb[slot, 0, pl.ds(g * 16, 16)]
                z1 = zb[slot, 1, pl.ds(g * 16, 16)]
                o0 = plsc.load_gather(acc_a, [dv])
                n0 = jnp.maximum(o0, z0)
                plsc.store_scatter(acc_a, [dv], n0)
                o1 = plsc.load_gather(acc_b, [dv])
                n1 = jnp.maximum(o1, z1)
                plsc.store_scatter(acc_b, [dv], n1)
                c0 = plsc.load_gather(acc_a, [dv])
                c1 = plsc.load_gather(acc_b, [dv])
                p0 = (c0 < n0).astype(I32)
                p1 = (c1 < n1).astype(I32)

                def cond(st):
                    q0, q1 = st
                    return jnp.max(q0 + q1) > 0

                def wbody(st):
                    q0, q1 = st
                    m0 = q0 > 0
                    m1 = q1 > 0
                    a0 = plsc.load_gather(acc_a, [dv])
                    u0 = jnp.maximum(a0, z0)
                    plsc.store_scatter(acc_a, [dv], u0, mask=m0)
                    a1 = plsc.load_gather(acc_b, [dv])
                    u1 = jnp.maximum(a1, z1)
                    plsc.store_scatter(acc_b, [dv], u1, mask=m1)
                    r0 = plsc.load_gather(acc_a, [dv])
                    r1 = plsc.load_gather(acc_b, [dv])
                    return (jnp.logical_and(m0, r0 < u0).astype(I32),
                            jnp.logical_and(m1, r1 < u1).astype(I32))

                lax.while_loop(cond, wbody, (p0, p1))
                return carry

            lax.fori_loop(0, 128, group, 0)

        fetch(0, 0)

        def it(ci, slot):
            nxt = 1 - slot

            @pl.when(ci + 1 < nc)
            def _():
                fetch(ci + 1, nxt)

            fwait(slot)
            process(slot)

        def loop_body(ci, carry):
            @pl.when(ci % 2 == 0)
            def _():
                it(ci, 0)

            @pl.when(ci % 2 == 1)
            def _():
                it(ci, 1)

            return carry

        lax.fori_loop(0, nc, loop_body, 0)
        pltpu.sync_copy(acc_a.at[pl.ds(0, n_out)],
                        out_ref.at[pl.ds(2 * w * n_out, n_out)])
        pltpu.sync_copy(acc_b.at[pl.ds(0, n_out)],
                        out_ref.at[pl.ds((2 * w + 1) * n_out, n_out)])

    f = functools.partial(
        pl.kernel,
        out_type=jax.ShapeDtypeStruct((H * n_out,), F32),
        mesh=_mesh(),
        compiler_params=pltpu.CompilerParams(use_tc_tiling_on_sc=False, needs_layout_passes=False),
        scratch_types=[
            pltpu.VMEM((2, 2, 2048), F32),
            pltpu.VMEM((2, 2048), I32),
            pltpu.VMEM((n_acc,), F32),
            pltpu.VMEM((n_acc,), F32),
            pltpu.SemaphoreType.DMA,
            pltpu.SemaphoreType.DMA,
        ],
    )(body)
    return f(zt.reshape(-1), dst).reshape(H, n_out)


def _spec(block, imap):
    return pl.BlockSpec(block, imap)


def _tc_ffn(part, l1c, W1, b1, W2, b2):
    def body(p_ref, c_ref, w1, bb1, w2, bb2, o_ref):
        s = p_ref[0] + p_ref[1]
        agg = jnp.concatenate(
            [s[:, 0:1], s[:, 1:4] - s[:, 4:5] * c_ref[...]], axis=1)
        t = jnp.maximum(jnp.dot(agg, w1[...]) + bb1[...], 0.0)
        o_ref[...] = jnp.maximum(jnp.dot(t, w2[...]) + bb2[...], 0.0)

    return pl.pallas_call(
        body,
        grid=(P1 // 512,),
        in_specs=[
            _spec((2, 512, 16), lambda i: (0, i, 0)),
            _spec((512, 3), lambda i: (i, 0)),
            _spec((4, H), lambda i: (0, 0)),
            _spec((1, H), lambda i: (0, 0)),
            _spec((H, H), lambda i: (0, 0)),
            _spec((1, H), lambda i: (0, 0)),
        ],
        out_specs=_spec((512, H), lambda i: (i, 0)),
        out_shape=jax.ShapeDtypeStruct((P1, H), F32),
    )(part, l1c, W1, b1, W2, b2)


def _tc_gnnpre(x, c, W1x, W1c, b1, n):
    def body(x_ref, c_ref, wx, wc, bb, a_ref, b_ref):
        cc = jnp.dot(c_ref[...], wc[...])
        b_ref[...] = cc
        a_ref[...] = jnp.dot(x_ref[...], wx[...]) + cc + bb[...]

    return pl.pallas_call(
        body,
        grid=(n // 512,),
        in_specs=[
            _spec((512, H), lambda i: (i, 0)),
            _spec((512, 3), lambda i: (i, 0)),
            _spec((H, H), lambda i: (0, 0)),
            _spec((3, H), lambda i: (0, 0)),
            _spec((1, H), lambda i: (0, 0)),
        ],
        out_specs=(_spec((512, H), lambda i: (i, 0)),
                   _spec((512, H), lambda i: (i, 0))),
        out_shape=(jax.ShapeDtypeStruct((n, H), F32),
                   jax.ShapeDtypeStruct((n, H), F32)),
    )(x, c, W1x, W1c, b1)


def _tc_edge(S, D, W2, b2c, Ep):
    def body(s_ref, d_ref, w2, bb, o_ref):
        h = jnp.maximum(s_ref[...] - d_ref[...], 0.0)
        zt = lax.dot_general(w2[...], h, (((0,), (1,)), ((), ())))
        o_ref[...] = jnp.maximum(zt + bb[...], 0.0)

    return pl.pallas_call(
        body,
        grid=(Ep // 512,),
        in_specs=[
            _spec((512, H), lambda i: (i, 0)),
            _spec((512, H), lambda i: (i, 0)),
            _spec((H, H), lambda i: (0, 0)),
            _spec((H, 1), lambda i: (0, 0)),
        ],
        out_specs=_spec((H, 512), lambda i: (0, i)),
        out_shape=jax.ShapeDtypeStruct((H, Ep), F32),
    )(S, D, W2, b2c)


def _tc_post(aggT, x, W4, b4, W5, b5, n, x2=None):
    def body(*refs):
        if x2 is None:
            a_ref, x_ref, w4, bb4, w5, bb5, o_ref = refs
        else:
            a_ref, x_ref, x2_ref, w4, bb4, w5, bb5, o_ref = refs
        a4 = lax.dot_general(a_ref[...], w4[...], (((0,), (0,)), ((), ())))
        y1 = jnp.maximum(a4 + bb4[...], 0.0)
        y2 = jnp.maximum(jnp.dot(y1, w5[...]) + bb5[...], 0.0)
        o = x_ref[...] + y2
        if x2 is not None:
            o = o + x2_ref[...]
        o_ref[...] = o

    in_specs = [_spec((H, 512), lambda i: (0, i)),
                _spec((512, H), lambda i: (i, 0))]
    args = [aggT, x]
    if x2 is not None:
        in_specs.append(_spec((512, H), lambda i: (i, 0)))
        args.append(x2)
    in_specs += [
        _spec((H, H), lambda i: (0, 0)),
        _spec((1, H), lambda i: (0, 0)),
        _spec((H, H), lambda i: (0, 0)),
        _spec((1, H), lambda i: (0, 0)),
    ]
    args += [W4, b4, W5, b5]
    return pl.pallas_call(
        body,
        grid=(n // 512,),
        in_specs=in_specs,
        out_specs=_spec((512, H), lambda i: (i, 0)),
        out_shape=jax.ShapeDtypeStruct((n, H), F32),
    )(*args)


def _tc_cw(c, Wb, n):
    def body(c_ref, wb, o_ref):
        o_ref[...] = jnp.dot(c_ref[...], wb[...])

    return pl.pallas_call(
        body,
        grid=(n // 512,),
        in_specs=[_spec((512, 3), lambda i: (i, 0)),
                  _spec((3, H), lambda i: (0, 0))],
        out_specs=_spec((512, H), lambda i: (i, 0)),
        out_shape=jax.ShapeDtypeStruct((n, H), F32),
    )(c, Wb)


def _tc_m2l(t2p, l1cp, gp, Wa, Wb, bc, Np):
    def body2(t_ref, c_ref, g_ref, wa, wb, bb, o_ref):
        m = lax.dot_general(wa[...], t_ref[...], (((0,), (1,)), ((), ())))
        m = m + lax.dot_general(wb[...], c_ref[...], (((0,), (1,)), ((), ())))
        o_ref[...] = jnp.maximum(m - g_ref[...].T + bb[...], 0.0)

    return pl.pallas_call(
        body2,
        grid=(Np // 512,),
        in_specs=[
            _spec((512, H), lambda i: (i, 0)),
            _spec((512, 3), lambda i: (i, 0)),
            _spec((512, H), lambda i: (i, 0)),
            _spec((H, H), lambda i: (0, 0)),
            _spec((3, H), lambda i: (0, 0)),
            _spec((H, 1), lambda i: (0, 0)),
        ],
        out_specs=_spec((H, 512), lambda i: (0, i)),
        out_shape=jax.ShapeDtypeStruct((H, Np), F32),
    )(t2p, l1cp, gp, Wa, Wb, bc)


def _tc_trans(xT, n):
    def body(x_ref, o_ref):
        o_ref[...] = x_ref[...].T

    return pl.pallas_call(
        body,
        grid=(n // 512,),
        in_specs=[_spec((H, 512), lambda i: (0, i))],
        out_specs=_spec((512, H), lambda i: (i, 0)),
        out_shape=jax.ShapeDtypeStruct((n, H), F32),
    )(xT)


def _tc_l2mpre(t4, l2c, Wa, Wb, n):
    def body(t_ref, c_ref, wa, wb, o_ref):
        o_ref[...] = jnp.dot(t_ref[...], wa[...]) + jnp.dot(c_ref[...], wb[...])

    return pl.pallas_call(
        body,
        grid=(n // 512,),
        in_specs=[
            _spec((512, H), lambda i: (i, 0)),
            _spec((512, 3), lambda i: (i, 0)),
            _spec((H, H), lambda i: (0, 0)),
            _spec((3, H), lambda i: (0, 0)),
        ],
        out_specs=_spec((512, H), lambda i: (i, 0)),
        out_shape=jax.ShapeDtypeStruct((n, H), F32),
    )(t4, l2c, Wa, Wb)


def _tc_g6pre(gH, l1c, Wbl, bl, W1x, W1c, b1):
    def body(g_ref, c_ref, wbl, bbl, wx, wc, bb1, t5_ref, a_ref, b_ref):
        t5 = jnp.maximum(
            g_ref[...] - jnp.dot(c_ref[...], wbl[...]) + bbl[...], 0.0)
        t5_ref[...] = t5
        cc = jnp.dot(c_ref[...], wc[...])
        b_ref[...] = cc
        a_ref[...] = jnp.dot(t5, wx[...]) + cc + bb1[...]

    return pl.pallas_call(
        body,
        grid=(P1 // 512,),
        in_specs=[
            _spec((512, H), lambda i: (i, 0)),
            _spec((512, 3), lambda i: (i, 0)),
            _spec((3, H), lambda i: (0, 0)),
            _spec((1, H), lambda i: (0, 0)),
            _spec((H, H), lambda i: (0, 0)),
            _spec((3, H), lambda i: (0, 0)),
            _spec((1, H), lambda i: (0, 0)),
        ],
        out_specs=(_spec((512, H), lambda i: (i, 0)),
                   _spec((512, H), lambda i: (i, 0)),
                   _spec((512, H), lambda i: (i, 0))),
        out_shape=(jax.ShapeDtypeStruct((P1, H), F32),
                   jax.ShapeDtypeStruct((P1, H), F32),
                   jax.ShapeDtypeStruct((P1, H), F32)),
    )(gH, l1c, Wbl, bl, W1x, W1c, b1)


def _tc_fbnpre(t6, l1c, Wfa, Wfb, bf):
    def body(t_ref, c_ref, wa, wb, bb, o_ref):
        o_ref[...] = (jnp.dot(t_ref[...], wa[...])
                      - jnp.dot(c_ref[...], wb[...]) + bb[...])

    return pl.pallas_call(
        body,
        grid=(P1 // 512,),
        in_specs=[
            _spec((512, H), lambda i: (i, 0)),
            _spec((512, 3), lambda i: (i, 0)),
            _spec((H, H), lambda i: (0, 0)),
            _spec((3, H), lambda i: (0, 0)),
            _spec((1, H), lambda i: (0, 0)),
        ],
        out_specs=_spec((512, H), lambda i: (i, 0)),
        out_shape=jax.ShapeDtypeStruct((P1, H), F32),
    )(t6, l1c, Wfa, Wfb, bf)


def _tc_final(gG, pts, rem, Wfb, Wfc, Wc, bc):
    def body(g_ref, p_ref, r_ref, wb, wc1, wcls, bcls, o_ref):
        t7 = jnp.maximum(
            g_ref[...] + jnp.dot(p_ref[...], wb[...])
            + jnp.dot(r_ref[...], wc1[...]), 0.0)
        o_ref[...] = jnp.dot(t7, wcls[...]) + bcls[...]

    return pl.pallas_call(
        body,
        grid=(N // 1000,),
        in_specs=[
            _spec((1000, H), lambda i: (i, 0)),
            _spec((1000, 3), lambda i: (i, 0)),
            _spec((1000, 1), lambda i: (i, 0)),
            _spec((3, H), lambda i: (0, 0)),
            _spec((1, H), lambda i: (0, 0)),
            _spec((H, NCLS), lambda i: (0, 0)),
            _spec((1, NCLS), lambda i: (0, 0)),
        ],
        out_specs=_spec((1000, NCLS), lambda i: (i, 0)),
        out_shape=jax.ShapeDtypeStruct((N, NCLS), F32),
    )(gG, pts, rem, Wfb, Wfc, Wc, bc)


def _pad_idx(idx, Bp, fill):
    p = jnp.pad(idx.astype(I32), (0, Bp - idx.shape[0]), constant_values=fill)
    return p


def _gnn_layer(x, centers, edges, p, npad, ntrash, Bp, n_acc, x2=None):
    W1, b1, W2, b2, W4, b4, W5, b5 = p
    W1x, W1c = W1[:H], W1[H:]
    A, B = _tc_gnnpre(x, centers, W1x, W1c, b1.reshape(1, H), npad)
    src2 = _pad_idx(edges[0], Bp, 0).reshape(Bp // 128, 128)
    dst2 = _pad_idx(edges[1], Bp, 0).reshape(Bp // 128, 128)
    S = _sc_gather(A, src2, npad, Bp)
    D = _sc_gather(B, dst2, npad, Bp)
    zt = _tc_edge(S, D, W2, b2.reshape(H, 1), Bp)
    dsts = _pad_idx(edges[1], Bp, ntrash)
    aggT = _sc_smax(zt, dsts, npad, n_acc, Bp)
    return _tc_post(aggT, x, W4, b4.reshape(1, H), W5, b5.reshape(1, H), npad,
                    x2=x2)


def kernel(remission, points, l1_cluster_centers, l2_cluster_centers,
           l1_edges, l2_edges, l1_labels, l2_labels, params):
    l1c = jnp.pad(l1_cluster_centers, ((0, P1 - N1), (0, 0)))
    l2c = jnp.pad(l2_cluster_centers, ((0, P2 - N2), (0, 0)))
    l1_labels = l1_labels.astype(I32)
    l2_labels = l2_labels.astype(I32)
    l1_edges = l1_edges.astype(I32)
    l2_edges = l2_edges.astype(I32)

    NP_FFN = 131072
    BP1 = 425984
    BP2 = 98304
    BPL1 = 131072
    BPL2 = 32768
    NP_M2L = 26624

    pf = jnp.concatenate(
        [remission, points, jnp.ones((N, 1), F32), jnp.zeros((N, 11), F32)],
        axis=1)
    pf = jnp.pad(pf, ((0, NP_FFN - N), (0, 0)))
    lab2 = _pad_idx(l1_labels, NP_FFN, N1).reshape(NP_FFN // 128, 128)
    zero_acc = jnp.zeros((NACC1, 16), F32)
    part = _sc_ffn_add(pf, lab2, zero_acc, NP_FFN)
    fW1, fb1, fW2, fb2 = params["ffn"]
    t1 = _tc_ffn(part, l1c, fW1, fb1.reshape(1, H), fW2, fb2.reshape(1, H))

    t2 = _gnn_layer(t1, l1c, l1_edges, params["g2"], P1, N1, BP1, NACC1)

    mW, mb = params["m2l"]
    l2cw = _tc_cw(l2c, mW[H:], P2)
    g2l = _sc_gather(l2cw, _pad_idx(l2_labels, BPL2, 0).reshape(-1, 128),
                     P2, BPL2)
    t2p = jnp.pad(t2, ((0, NP_M2L - P1), (0, 0)))
    l1cp = jnp.pad(l1c, ((0, NP_M2L - P1), (0, 0)))
    gp = g2l[:NP_M2L]
    mfT = _tc_m2l(t2p, l1cp, gp, mW[:H], mW[H:], mb.reshape(H, 1), NP_M2L)
    mdst = _pad_idx(l2_labels, NP_M2L, N2)
    t3T = _sc_smax(mfT, mdst, P2, NACC2, NP_M2L)
    t3 = _tc_trans(t3T, P2)

    t4 = _gnn_layer(t3, l2c, l2_edges, params["g4"], P2, N2, BP2, NACC2)
    t4 = _gnn_layer(t4, l2c, l2_edges, params["g41"], P2, N2, BP2, NACC2)

    lW, lb = params["l2m"]
    H2 = _tc_l2mpre(t4, l2c, lW[:H], lW[H:], P2)
    gH = _sc_gather(H2, _pad_idx(l2_labels, BPL2, 0).reshape(-1, 128),
                    P2, BPL2)[:P1]
    g6W1, g6b1 = params["g6"][0], params["g6"][1]
    t5, A6, B6 = _tc_g6pre(gH, l1c, lW[H:], lb.reshape(1, H),
                           g6W1[:H], g6W1[H:], g6b1.reshape(1, H))
    src2 = _pad_idx(l1_edges[0], BP1, 0).reshape(-1, 128)
    dst2 = _pad_idx(l1_edges[1], BP1, 0).reshape(-1, 128)
    S6 = _sc_gather(A6, src2, P1, BP1)
    D6 = _sc_gather(B6, dst2, P1, BP1)
    zt6 = _tc_edge(S6, D6, params["g6"][2], params["g6"][3].reshape(H, 1), BP1)
    agg6T = _sc_smax(zt6, _pad_idx(l1_edges[1], BP1, N1), P1, NACC1, BP1)
    t6 = _tc_post(agg6T, t5, params["g6"][4], params["g6"][5].reshape(1, H),
                  params["g6"][6], params["g6"][7].reshape(1, H), P1, x2=t2)

    fbW, fbb = params["fbn"]
    G = _tc_fbnpre(t6, l1c, fbW[:H], fbW[H:H + 3], fbb.reshape(1, H))
    gG = _sc_gather(G, _pad_idx(l1_labels, BPL1, 0).reshape(-1, 128),
                    P1, BPL1)[:N]
    cW, cb = params["cls"]
    return _tc_final(gG, points, remission, fbW[H:H + 3], fbW[H + 3:],
                     cW, cb.reshape(1, NCLS))


# speedup vs baseline: 1.3419x; 1.3419x over previous
import functools

import jax
import jax.numpy as jnp
from jax import lax
from jax.experimental import pallas as pl
from jax.experimental.pallas import tpu as pltpu
from jax.experimental.pallas import tpu_sc as plsc

F32 = jnp.float32
I32 = jnp.int32
N, N1, N2 = 100000, 25000, 5000
E1, E2 = 400000, 80000
H = 64
NCLS = 20
NWORK = 32
P1 = 25600
P2 = 5120
NACC1 = P1 + 128
NACC2 = P2 + 128


def _mesh():
    return plsc.VectorSubcoreMesh(core_axis_name="c", subcore_axis_name="s")


def _wid():
    return lax.axis_index("s") * 2 + lax.axis_index("c")


def _sc_gather(table, idx2, V, Bp, D=H):
    cpw = Bp // NWORK
    nb = cpw // 512
    kpc = cpw // 128
    assert cpw % 512 == 0

    def body(tab_ref, idx_ref, out_ref, idx_v, bufs, g0, g1, w0, w1):
        w = _wid()
        pltpu.sync_copy(idx_ref.at[pl.ds(w * kpc, kpc)], idx_v)
        gsems = (g0, g1)
        wsems = (w0, w1)

        def fire(b, slot):
            for j in range(4):
                pltpu.make_async_copy(
                    tab_ref.at[idx_v.at[b * 4 + j]],
                    bufs.at[slot, pl.ds(j * 128, 128)],
                    gsems[slot]).start()

        def gwait(slot):
            pltpu.make_async_copy(
                tab_ref.at[pl.ds(0, 512)], bufs.at[slot], gsems[slot]).wait()

        def wstart(b, slot):
            pltpu.async_copy(
                bufs.at[slot], out_ref.at[pl.ds(w * cpw + b * 512, 512)],
                wsems[slot])

        def wwait(slot):
            pltpu.make_async_copy(
                bufs.at[slot], out_ref.at[pl.ds(0, 512)], wsems[slot]).wait()

        fire(0, 0)

        def it(b, slot):
            nxt = 1 - slot

            @pl.when(b >= 1)
            def _():
                wwait(nxt)

            @pl.when(b + 1 < nb)
            def _():
                fire(b + 1, nxt)

            gwait(slot)
            wstart(b, slot)

        def loop_body(b, carry):
            @pl.when(b % 2 == 0)
            def _():
                it(b, 0)

            @pl.when(b % 2 == 1)
            def _():
                it(b, 1)

            return carry

        lax.fori_loop(0, nb, loop_body, 0)
        wwait((nb - 1) % 2)

    f = functools.partial(
        pl.kernel,
        out_type=jax.ShapeDtypeStruct((Bp, D), F32),
        mesh=_mesh(),
        compiler_params=pltpu.CompilerParams(use_tc_tiling_on_sc=False,
                                             needs_layout_passes=False),
        scratch_types=[
            pltpu.VMEM((kpc, 128), I32),
            pltpu.VMEM((2, 512, D), F32),
            pltpu.SemaphoreType.DMA,
            pltpu.SemaphoreType.DMA,
            pltpu.SemaphoreType.DMA,
            pltpu.SemaphoreType.DMA,
        ],
    )(body)
    return f(table, idx2)


def _sc_gather_sc(tabA, tabC, srcI, dstI, V, Bp):
    cpw = Bp // NWORK
    nb = cpw // 512
    kpc = cpw // 128
    assert cpw % 512 == 0

    def body(ta_ref, tc_ref, si_ref, di_ref, outS, outC,
             six, dix, bufS, bufC, g0, g1, w0, w1):
        w = _wid()
        pltpu.sync_copy(si_ref.at[pl.ds(w * kpc, kpc)], six)
        pltpu.sync_copy(di_ref.at[pl.ds(w * kpc, kpc)], dix)
        gsems = (g0, g1)
        wsems = (w0, w1)

        def fire(b, slot):
            for j in range(4):
                pltpu.make_async_copy(
                    ta_ref.at[six.at[b * 4 + j]],
                    bufS.at[slot, pl.ds(j * 128, 128)],
                    gsems[slot]).start()
                pltpu.make_async_copy(
                    tc_ref.at[dix.at[b * 4 + j]],
                    bufC.at[slot, pl.ds(j * 128, 128)],
                    gsems[slot]).start()

        def gwait(slot):
            pltpu.make_async_copy(
                ta_ref.at[pl.ds(0, 512)], bufS.at[slot], gsems[slot]).wait()
            pltpu.make_async_copy(
                tc_ref.at[pl.ds(0, 512)], bufC.at[slot], gsems[slot]).wait()

        def wstart(b, slot):
            pltpu.async_copy(
                bufS.at[slot], outS.at[pl.ds(w * cpw + b * 512, 512)],
                wsems[slot])
            pltpu.async_copy(
                bufC.at[slot], outC.at[pl.ds(w * cpw + b * 512, 512)],
                wsems[slot])

        def wwait(slot):
            pltpu.make_async_copy(
                bufS.at[slot], outS.at[pl.ds(0, 512)], wsems[slot]).wait()
            pltpu.make_async_copy(
                bufC.at[slot], outC.at[pl.ds(0, 512)], wsems[slot]).wait()

        fire(0, 0)

        def it(b, slot):
            nxt = 1 - slot

            @pl.when(b >= 1)
            def _():
                wwait(nxt)

            @pl.when(b + 1 < nb)
            def _():
                fire(b + 1, nxt)

            gwait(slot)
            wstart(b, slot)

        def loop_body(b, carry):
            @pl.when(b % 2 == 0)
            def _():
                it(b, 0)

            @pl.when(b % 2 == 1)
            def _():
                it(b, 1)

            return carry

        lax.fori_loop(0, nb, loop_body, 0)
        wwait((nb - 1) % 2)

    f = functools.partial(
        pl.kernel,
        out_type=(jax.ShapeDtypeStruct((Bp, H), F32),
                  jax.ShapeDtypeStruct((Bp, 16), F32)),
        mesh=_mesh(),
        compiler_params=pltpu.CompilerParams(use_tc_tiling_on_sc=False,
                                             needs_layout_passes=False),
        scratch_types=[
            pltpu.VMEM((kpc, 128), I32),
            pltpu.VMEM((kpc, 128), I32),
            pltpu.VMEM((2, 512, H), F32),
            pltpu.VMEM((2, 512, 16), F32),
            pltpu.SemaphoreType.DMA,
            pltpu.SemaphoreType.DMA,
            pltpu.SemaphoreType.DMA,
            pltpu.SemaphoreType.DMA,
        ],
    )(body)
    return f(tabA, tabC, srcI, dstI)


def _sc_ffn_add(pf, lab2, zero_acc, Np):
    cpw = Np // NWORK
    kpc = cpw // 128
    nacc = NACC1

    def body(pf_ref, lab_ref, zero_ref, out_ref, idx_v, rows_v, acc):
        w = _wid()
        s = lax.axis_index("s")
        c = lax.axis_index("c")
        stripe = nacc // 16
        pltpu.sync_copy(zero_ref.at[pl.ds(s * stripe, stripe)],
                        acc.at[pl.ds(s * stripe, stripe)])
        plsc.subcore_barrier()
        pltpu.sync_copy(lab_ref.at[pl.ds(w * kpc, kpc)], idx_v)

        def loop_body(j, carry):
            pltpu.sync_copy(pf_ref.at[pl.ds(w * cpw + j * 128, 128)], rows_v)
            pltpu.sync_copy(rows_v, acc.at[idx_v.at[j]], add=True)
            return carry

        lax.fori_loop(0, kpc, loop_body, 0)
        plsc.subcore_barrier()

        @pl.when(s == 0)
        def _():
            pltpu.sync_copy(acc.at[pl.ds(0, P1)], out_ref.at[c])

    f = functools.partial(
        pl.kernel,
        out_type=jax.ShapeDtypeStruct((2, P1, 16), F32),
        mesh=_mesh(),
        compiler_params=pltpu.CompilerParams(use_tc_tiling_on_sc=False, needs_layout_passes=False),
        scratch_types=[
            pltpu.VMEM((kpc, 128), I32),
            pltpu.VMEM((128, 16), F32),
            pltpu.VMEM_SHARED((nacc, 16), F32),
        ],
    )(body)
    return f(pf, lab2, zero_acc)


def _sc_smax(zt, dst, n_out, n_acc, Ep):
    nc2 = Ep // 2048
    assert Ep % 2048 == 0

    def body(zt_ref, dst_ref, out_ref, zb, db, a0, a1, a2, a3, f0, f1):
        w = _wid()
        fg = w // 2
        half = w % 2
        accs = (a0, a1, a2, a3)

        def zero_body(i, carry):
            z16 = jnp.zeros((16,), F32)
            for a in accs:
                a[pl.ds(i * 16, 16)] = z16
            return carry

        lax.fori_loop(0, n_acc // 16, zero_body, 0)
        fsems = (f0, f1)
        ebase = half * nc2 * 1024

        def fetch(ci, slot):
            for f in range(4):
                pltpu.make_async_copy(
                    zt_ref.at[pl.ds((4 * fg + f) * Ep + ebase + ci * 1024,
                                    1024)],
                    zb.at[slot, f], fsems[slot]).start()
            pltpu.make_async_copy(
                dst_ref.at[pl.ds(ebase + ci * 1024, 1024)], db.at[slot],
                fsems[slot]).start()

        def fwait(slot):
            for f in range(4):
                pltpu.make_async_copy(
                    zt_ref.at[pl.ds(0, 1024)], zb.at[slot, f],
                    fsems[slot]).wait()
            pltpu.make_async_copy(
                dst_ref.at[pl.ds(0, 1024)], db.at[slot], fsems[slot]).wait()

        def process(slot):
            def group(g, carry):
                dv = db[slot, pl.ds(g * 16, 16)]
                zs = [zb[slot, f, pl.ds(g * 16, 16)] for f in range(4)]
                ns = []
                for f in range(4):
                    o = plsc.load_gather(accs[f], [dv])
                    nv = jnp.maximum(o, zs[f])
                    plsc.store_scatter(accs[f], [dv], nv)
                    ns.append(nv)
                pend = jnp.zeros((16,), I32)
                for f in range(4):
                    c = plsc.load_gather(accs[f], [dv])
                    pend = pend + (c < ns[f]).astype(I32)

                def cond(q):
                    return jnp.max(q) > 0

                def wbody(q):
                    m = q > 0
                    nq = jnp.zeros((16,), I32)
                    for f in range(4):
                        a = plsc.load_gather(accs[f], [dv])
                        u = jnp.maximum(a, zs[f])
                        plsc.store_scatter(accs[f], [dv], u, mask=m)
                        r = plsc.load_gather(accs[f], [dv])
                        nq = nq + jnp.logical_and(m, r < u).astype(I32)
                    return nq

                lax.while_loop(cond, wbody, pend)
                return carry

            lax.fori_loop(0, 64, group, 0)

        fetch(0, 0)

        def it(ci, slot):
            nxt = 1 - slot

            @pl.when(ci + 1 < nc2)
            def _():
                fetch(ci + 1, nxt)

            fwait(slot)
            process(slot)

        def loop_body(ci, carry):
            @pl.when(ci % 2 == 0)
            def _():
                it(ci, 0)

            @pl.when(ci % 2 == 1)
            def _():
                it(ci, 1)

            return carry

        lax.fori_loop(0, nc2, loop_body, 0)
        for f in range(4):
            pltpu.sync_copy(
                accs[f].at[pl.ds(0, n_out)],
                out_ref.at[pl.ds((half * H + 4 * fg + f) * n_out, n_out)])

    f = functools.partial(
        pl.kernel,
        out_type=jax.ShapeDtypeStruct((2 * H * n_out,), F32),
        mesh=_mesh(),
        compiler_params=pltpu.CompilerParams(use_tc_tiling_on_sc=False,
                                             needs_layout_passes=False),
        scratch_types=[
            pltpu.VMEM((2, 4, 1024), F32),
            pltpu.VMEM((2, 1024), I32),
            pltpu.VMEM((n_acc,), F32),
            pltpu.VMEM((n_acc,), F32),
            pltpu.VMEM((n_acc,), F32),
            pltpu.VMEM((n_acc,), F32),
            pltpu.SemaphoreType.DMA,
            pltpu.SemaphoreType.DMA,
        ],
    )(body)
    return f(zt.reshape(-1), dst).reshape(2, H, n_out)


def _spec(block, imap):
    return pl.BlockSpec(block, imap)


def _tc_ffn(part, l1c, W1, b1, W2, b2):
    def body(p_ref, c_ref, w1, bb1, w2, bb2, o_ref):
        s = p_ref[0] + p_ref[1]
        agg = jnp.concatenate(
            [s[:, 0:1], s[:, 1:4] - s[:, 4:5] * c_ref[...]], axis=1)
        t = jnp.maximum(jnp.dot(agg, w1[...]) + bb1[...], 0.0)
        o_ref[...] = jnp.maximum(jnp.dot(t, w2[...]) + bb2[...], 0.0)

    return pl.pallas_call(
        body,
        grid=(P1 // 512,),
        in_specs=[
            _spec((2, 512, 16), lambda i: (0, i, 0)),
            _spec((512, 3), lambda i: (i, 0)),
            _spec((4, H), lambda i: (0, 0)),
            _spec((1, H), lambda i: (0, 0)),
            _spec((H, H), lambda i: (0, 0)),
            _spec((1, H), lambda i: (0, 0)),
        ],
        out_specs=_spec((512, H), lambda i: (i, 0)),
        out_shape=jax.ShapeDtypeStruct((P1, H), F32),
    )(part, l1c, W1, b1, W2, b2)


def _tc_gnnpre(x, c, W1x, W1c, b1, n):
    def body(x_ref, c_ref, wx, wc, bb, a_ref):
        a_ref[...] = (jnp.dot(x_ref[...], wx[...])
                      + jnp.dot(c_ref[...], wc[...]) + bb[...])

    return pl.pallas_call(
        body,
        grid=(n // 512,),
        in_specs=[
            _spec((512, H), lambda i: (i, 0)),
            _spec((512, 3), lambda i: (i, 0)),
            _spec((H, H), lambda i: (0, 0)),
            _spec((3, H), lambda i: (0, 0)),
            _spec((1, H), lambda i: (0, 0)),
        ],
        out_specs=_spec((512, H), lambda i: (i, 0)),
        out_shape=jax.ShapeDtypeStruct((n, H), F32),
    )(x, c, W1x, W1c, b1)


def _tc_edge(S, C16, W1c16, W2, b2c, Ep):
    def body(s_ref, c_ref, wc, w2, bb, o_ref):
        h = jnp.maximum(s_ref[...] - jnp.dot(c_ref[...], wc[...]), 0.0)
        zt = lax.dot_general(w2[...], h, (((0,), (1,)), ((), ())))
        o_ref[...] = jnp.maximum(zt + bb[...], 0.0)

    return pl.pallas_call(
        body,
        grid=(Ep // 512,),
        in_specs=[
            _spec((512, H), lambda i: (i, 0)),
            _spec((512, 16), lambda i: (i, 0)),
            _spec((16, H), lambda i: (0, 0)),
            _spec((H, H), lambda i: (0, 0)),
            _spec((H, 1), lambda i: (0, 0)),
        ],
        out_specs=_spec((H, 512), lambda i: (0, i)),
        out_shape=jax.ShapeDtypeStruct((H, Ep), F32),
    )(S, C16, W1c16, W2, b2c)


def _tc_post(aggT, x, W4, b4, W5, b5, n, x2=None):
    def body(*refs):
        if x2 is None:
            a_ref, x_ref, w4, bb4, w5, bb5, o_ref = refs
        else:
            a_ref, x_ref, x2_ref, w4, bb4, w5, bb5, o_ref = refs
        a = jnp.maximum(a_ref[0], a_ref[1])
        a4 = lax.dot_general(a, w4[...], (((0,), (0,)), ((), ())))
        y1 = jnp.maximum(a4 + bb4[...], 0.0)
        y2 = jnp.maximum(jnp.dot(y1, w5[...]) + bb5[...], 0.0)
        o = x_ref[...] + y2
        if x2 is not None:
            o = o + x2_ref[...]
        o_ref[...] = o

    in_specs = [_spec((2, H, 512), lambda i: (0, 0, i)),
                _spec((512, H), lambda i: (i, 0))]
    args = [aggT, x]
    if x2 is not None:
        in_specs.append(_spec((512, H), lambda i: (i, 0)))
        args.append(x2)
    in_specs += [
        _spec((H, H), lambda i: (0, 0)),
        _spec((1, H), lambda i: (0, 0)),
        _spec((H, H), lambda i: (0, 0)),
        _spec((1, H), lambda i: (0, 0)),
    ]
    args += [W4, b4, W5, b5]
    return pl.pallas_call(
        body,
        grid=(n // 512,),
        in_specs=in_specs,
        out_specs=_spec((512, H), lambda i: (i, 0)),
        out_shape=jax.ShapeDtypeStruct((n, H), F32),
    )(*args)


def _tc_m2l(t2p, l1cp, cgp, Wa, Wb, bc, Np):
    def body(t_ref, c_ref, g_ref, wa, wb, bb, o_ref):
        m = lax.dot_general(wa[...], t_ref[...], (((0,), (1,)), ((), ())))
        d3 = c_ref[...] - g_ref[...][:, :3]
        m = m + lax.dot_general(wb[...], d3, (((0,), (1,)), ((), ())))
        o_ref[...] = jnp.maximum(m + bb[...], 0.0)

    return pl.pallas_call(
        body,
        grid=(Np // 512,),
        in_specs=[
            _spec((512, H), lambda i: (i, 0)),
            _spec((512, 3), lambda i: (i, 0)),
            _spec((512, 16), lambda i: (i, 0)),
            _spec((H, H), lambda i: (0, 0)),
            _spec((3, H), lambda i: (0, 0)),
            _spec((H, 1), lambda i: (0, 0)),
        ],
        out_specs=_spec((H, 512), lambda i: (0, i)),
        out_shape=jax.ShapeDtypeStruct((H, Np), F32),
    )(t2p, l1cp, cgp, Wa, Wb, bc)


def _tc_trans(xT, n):
    def body(x_ref, o_ref):
        o_ref[...] = jnp.maximum(x_ref[0], x_ref[1]).T

    return pl.pallas_call(
        body,
        grid=(n // 512,),
        in_specs=[_spec((2, H, 512), lambda i: (0, 0, i))],
        out_specs=_spec((512, H), lambda i: (i, 0)),
        out_shape=jax.ShapeDtypeStruct((n, H), F32),
    )(xT)


def _tc_l2mpre(t4, l2c, Wa, Wb, n):
    def body(t_ref, c_ref, wa, wb, o_ref):
        o_ref[...] = jnp.dot(t_ref[...], wa[...]) + jnp.dot(c_ref[...], wb[...])

    return pl.pallas_call(
        body,
        grid=(n // 512,),
        in_specs=[
            _spec((512, H), lambda i: (i, 0)),
            _spec((512, 3), lambda i: (i, 0)),
            _spec((H, H), lambda i: (0, 0)),
            _spec((3, H), lambda i: (0, 0)),
        ],
        out_specs=_spec((512, H), lambda i: (i, 0)),
        out_shape=jax.ShapeDtypeStruct((n, H), F32),
    )(t4, l2c, Wa, Wb)


def _tc_g6pre(gH, l1c, Wbl, bl, W1x, W1c, b1):
    def body(g_ref, c_ref, wbl, bbl, wx, wc, bb1, t5_ref, a_ref):
        t5 = jnp.maximum(
            g_ref[...] - jnp.dot(c_ref[...], wbl[...]) + bbl[...], 0.0)
        t5_ref[...] = t5
        a_ref[...] = (jnp.dot(t5, wx[...]) + jnp.dot(c_ref[...], wc[...])
                      + bb1[...])

    return pl.pallas_call(
        body,
        grid=(P1 // 512,),
        in_specs=[
            _spec((512, H), lambda i: (i, 0)),
            _spec((512, 3), lambda i: (i, 0)),
            _spec((3, H), lambda i: (0, 0)),
            _spec((1, H), lambda i: (0, 0)),
            _spec((H, H), lambda i: (0, 0)),
            _spec((3, H), lambda i: (0, 0)),
            _spec((1, H), lambda i: (0, 0)),
        ],
        out_specs=(_spec((512, H), lambda i: (i, 0)),
                   _spec((512, H), lambda i: (i, 0))),
        out_shape=(jax.ShapeDtypeStruct((P1, H), F32),
                   jax.ShapeDtypeStruct((P1, H), F32)),
    )(gH, l1c, Wbl, bl, W1x, W1c, b1)


def _tc_fbnpre(t6, l1c, Wfa, Wfb, bf):
    def body(t_ref, c_ref, wa, wb, bb, o_ref):
        o_ref[...] = (jnp.dot(t_ref[...], wa[...])
                      - jnp.dot(c_ref[...], wb[...]) + bb[...])

    return pl.pallas_call(
        body,
        grid=(P1 // 512,),
        in_specs=[
            _spec((512, H), lambda i: (i, 0)),
            _spec((512, 3), lambda i: (i, 0)),
            _spec((H, H), lambda i: (0, 0)),
            _spec((3, H), lambda i: (0, 0)),
            _spec((1, H), lambda i: (0, 0)),
        ],
        out_specs=_spec((512, H), lambda i: (i, 0)),
        out_shape=jax.ShapeDtypeStruct((P1, H), F32),
    )(t6, l1c, Wfa, Wfb, bf)


def _tc_final(gG, pts, rem, Wfb, Wfc, Wc, bc):
    def body(g_ref, p_ref, r_ref, wb, wc1, wcls, bcls, o_ref):
        t7 = jnp.maximum(
            g_ref[...] + jnp.dot(p_ref[...], wb[...])
            + jnp.dot(r_ref[...], wc1[...]), 0.0)
        o_ref[...] = jnp.dot(t7, wcls[...]) + bcls[...]

    return pl.pallas_call(
        body,
        grid=(N // 1000,),
        in_specs=[
            _spec((1000, H), lambda i: (i, 0)),
            _spec((1000, 3), lambda i: (i, 0)),
            _spec((1000, 1), lambda i: (i, 0)),
            _spec((3, H), lambda i: (0, 0)),
            _spec((1, H), lambda i: (0, 0)),
            _spec((H, NCLS), lambda i: (0, 0)),
            _spec((1, NCLS), lambda i: (0, 0)),
        ],
        out_specs=_spec((1000, NCLS), lambda i: (i, 0)),
        out_shape=jax.ShapeDtypeStruct((N, NCLS), F32),
    )(gG, pts, rem, Wfb, Wfc, Wc, bc)


def _pad_idx(idx, Bp, fill):
    p = jnp.pad(idx.astype(I32), (0, Bp - idx.shape[0]), constant_values=fill)
    return p


def _gnn_core(A, c16, edges, npad, ntrash, Bp, n_acc, W1c, W2, b2):
    src2 = _pad_idx(edges[0], Bp, 0).reshape(Bp // 128, 128)
    dst2 = _pad_idx(edges[1], Bp, 0).reshape(Bp // 128, 128)
    S, C = _sc_gather_sc(A, c16, src2, dst2, npad, Bp)
    W1c16 = jnp.pad(W1c, ((0, 13), (0, 0)))
    zt = _tc_edge(S, C, W1c16, W2, b2.reshape(H, 1), Bp)
    dsts = _pad_idx(edges[1], Bp, ntrash)
    return _sc_smax(zt, dsts, npad, n_acc, Bp)


def _gnn_layer(x, centers, c16, edges, p, npad, ntrash, Bp, n_acc, x2=None):
    W1, b1, W2, b2, W4, b4, W5, b5 = p
    W1x, W1c = W1[:H], W1[H:]
    A = _tc_gnnpre(x, centers, W1x, W1c, b1.reshape(1, H), npad)
    aggT = _gnn_core(A, c16, edges, npad, ntrash, Bp, n_acc, W1c, W2, b2)
    return _tc_post(aggT, x, W4, b4.reshape(1, H), W5, b5.reshape(1, H), npad,
                    x2=x2)


def kernel(remission, points, l1_cluster_centers, l2_cluster_centers,
           l1_edges, l2_edges, l1_labels, l2_labels, params):
    l1c = jnp.pad(l1_cluster_centers, ((0, P1 - N1), (0, 0)))
    l2c = jnp.pad(l2_cluster_centers, ((0, P2 - N2), (0, 0)))
    l1_labels = l1_labels.astype(I32)
    l2_labels = l2_labels.astype(I32)
    l1_edges = l1_edges.astype(I32)
    l2_edges = l2_edges.astype(I32)

    NP_FFN = 131072
    BP1 = 425984
    BP2 = 98304
    BPL1 = 131072
    BPL2 = 32768
    NP_M2L = 26624

    pf = jnp.concatenate(
        [remission, points, jnp.ones((N, 1), F32), jnp.zeros((N, 11), F32)],
        axis=1)
    pf = jnp.pad(pf, ((0, NP_FFN - N), (0, 0)))
    lab2 = _pad_idx(l1_labels, NP_FFN, N1).reshape(NP_FFN // 128, 128)
    zero_acc = jnp.zeros((NACC1, 16), F32)
    part = _sc_ffn_add(pf, lab2, zero_acc, NP_FFN)
    fW1, fb1, fW2, fb2 = params["ffn"]
    t1 = _tc_ffn(part, l1c, fW1, fb1.reshape(1, H), fW2, fb2.reshape(1, H))

    l1c16 = jnp.pad(l1c, ((0, 0), (0, 13)))
    l2c16 = jnp.pad(l2c, ((0, 0), (0, 13)))
    t2 = _gnn_layer(t1, l1c, l1c16, l1_edges, params["g2"], P1, N1, BP1,
                    NACC1)

    mW, mb = params["m2l"]
    cg = _sc_gather(l2c16, _pad_idx(l2_labels, BPL2, 0).reshape(-1, 128),
                    P2, BPL2, D=16)
    t2p = jnp.pad(t2, ((0, NP_M2L - P1), (0, 0)))
    l1cp = jnp.pad(l1c, ((0, NP_M2L - P1), (0, 0)))
    cgp = cg[:NP_M2L]
    mfT = _tc_m2l(t2p, l1cp, cgp, mW[:H], mW[H:], mb.reshape(H, 1), NP_M2L)
    mdst = _pad_idx(l2_labels, NP_M2L, N2)
    t3T = _sc_smax(mfT, mdst, P2, NACC2, NP_M2L)
    t3 = _tc_trans(t3T, P2)

    t4 = _gnn_layer(t3, l2c, l2c16, l2_edges, params["g4"], P2, N2, BP2,
                    NACC2)
    t4 = _gnn_layer(t4, l2c, l2c16, l2_edges, params["g41"], P2, N2, BP2,
                    NACC2)

    lW, lb = params["l2m"]
    H2 = _tc_l2mpre(t4, l2c, lW[:H], lW[H:], P2)
    gH = _sc_gather(H2, _pad_idx(l2_labels, BPL2, 0).reshape(-1, 128),
                    P2, BPL2)[:P1]
    g6W1, g6b1 = params["g6"][0], params["g6"][1]
    t5, A6 = _tc_g6pre(gH, l1c, lW[H:], lb.reshape(1, H),
                       g6W1[:H], g6W1[H:], g6b1.reshape(1, H))
    agg6T = _gnn_core(A6, l1c16, l1_edges, P1, N1, BP1, NACC1,
                      g6W1[H:], params["g6"][2], params["g6"][3])
    t6 = _tc_post(agg6T, t5, params["g6"][4], params["g6"][5].reshape(1, H),
                  params["g6"][6], params["g6"][7].reshape(1, H), P1, x2=t2)

    fbW, fbb = params["fbn"]
    G = _tc_fbnpre(t6, l1c, fbW[:H], fbW[H:H + 3], fbb.reshape(1, H))
    gG = _sc_gather(G, _pad_idx(l1_labels, BPL1, 0).reshape(-1, 128),
                    P1, BPL1)[:N]
    cW, cb = params["cls"]
    return _tc_final(gG, points, remission, fbW[H:H + 3], fbW[H + 3:],
                     cW, cb.reshape(1, NCLS))


# speedup vs baseline: 1.4694x; 1.0950x over previous
import functools

import jax
import jax.numpy as jnp
from jax import lax
from jax.experimental import pallas as pl
from jax.experimental.pallas import tpu as pltpu
from jax.experimental.pallas import tpu_sc as plsc

F32 = jnp.float32
I32 = jnp.int32
N, N1, N2 = 100000, 25000, 5000
E1, E2 = 400000, 80000
H = 64
NCLS = 20
NWORK = 32
P1 = 25600
P2 = 5120
NACC1 = P1 + 128
NACC2 = P2 + 128


def _mesh():
    return plsc.VectorSubcoreMesh(core_axis_name="c", subcore_axis_name="s")


def _wid():
    return lax.axis_index("s") * 2 + lax.axis_index("c")


def _sc_gather(table, idx2, V, Bp, D=H):
    cpw = Bp // NWORK
    nb = cpw // 512
    kpc = cpw // 128
    assert cpw % 512 == 0

    def body(tab_ref, idx_ref, out_ref, idx_v, bufs, g0, g1, w0, w1):
        w = _wid()
        pltpu.sync_copy(idx_ref.at[pl.ds(w * kpc, kpc)], idx_v)
        gsems = (g0, g1)
        wsems = (w0, w1)

        def fire(b, slot):
            for j in range(4):
                pltpu.make_async_copy(
                    tab_ref.at[idx_v.at[b * 4 + j]],
                    bufs.at[slot, pl.ds(j * 128, 128)],
                    gsems[slot]).start()

        def gwait(slot):
            pltpu.make_async_copy(
                tab_ref.at[pl.ds(0, 512)], bufs.at[slot], gsems[slot]).wait()

        def wstart(b, slot):
            pltpu.async_copy(
                bufs.at[slot], out_ref.at[pl.ds(w * cpw + b * 512, 512)],
                wsems[slot])

        def wwait(slot):
            pltpu.make_async_copy(
                bufs.at[slot], out_ref.at[pl.ds(0, 512)], wsems[slot]).wait()

        fire(0, 0)

        def it(b, slot):
            nxt = 1 - slot

            @pl.when(b >= 1)
            def _():
                wwait(nxt)

            @pl.when(b + 1 < nb)
            def _():
                fire(b + 1, nxt)

            gwait(slot)
            wstart(b, slot)

        def loop_body(b, carry):
            @pl.when(b % 2 == 0)
            def _():
                it(b, 0)

            @pl.when(b % 2 == 1)
            def _():
                it(b, 1)

            return carry

        lax.fori_loop(0, nb, loop_body, 0)
        wwait((nb - 1) % 2)

    f = functools.partial(
        pl.kernel,
        out_type=jax.ShapeDtypeStruct((Bp, D), F32),
        mesh=_mesh(),
        compiler_params=pltpu.CompilerParams(use_tc_tiling_on_sc=False,
                                             needs_layout_passes=False),
        scratch_types=[
            pltpu.VMEM((kpc, 128), I32),
            pltpu.VMEM((2, 512, D), F32),
            pltpu.SemaphoreType.DMA,
            pltpu.SemaphoreType.DMA,
            pltpu.SemaphoreType.DMA,
            pltpu.SemaphoreType.DMA,
        ],
    )(body)
    return f(table, idx2)


def _sc_gather_sc(tabA, tabC, srcI, dstI, V, Bp):
    cpw = Bp // NWORK
    nb = cpw // 512
    kpc = cpw // 128
    assert cpw % 512 == 0

    def body(ta_ref, tc_ref, si_ref, di_ref, outS, outC,
             six, dix, bufS, bufC, g0, g1, w0, w1):
        w = _wid()
        pltpu.sync_copy(si_ref.at[pl.ds(w * kpc, kpc)], six)
        pltpu.sync_copy(di_ref.at[pl.ds(w * kpc, kpc)], dix)
        gsems = (g0, g1)
        wsems = (w0, w1)

        def fire(b, slot):
            for j in range(4):
                pltpu.make_async_copy(
                    ta_ref.at[six.at[b * 4 + j]],
                    bufS.at[slot, pl.ds(j * 128, 128)],
                    gsems[slot]).start()
                pltpu.make_async_copy(
                    tc_ref.at[dix.at[b * 4 + j]],
                    bufC.at[slot, pl.ds(j * 128, 128)],
                    gsems[slot]).start()

        def gwait(slot):
            pltpu.make_async_copy(
                ta_ref.at[pl.ds(0, 512)], bufS.at[slot], gsems[slot]).wait()
            pltpu.make_async_copy(
                tc_ref.at[pl.ds(0, 512)], bufC.at[slot], gsems[slot]).wait()

        def wstart(b, slot):
            pltpu.async_copy(
                bufS.at[slot], outS.at[pl.ds(w * cpw + b * 512, 512)],
                wsems[slot])
            pltpu.async_copy(
                bufC.at[slot], outC.at[pl.ds(w * cpw + b * 512, 512)],
                wsems[slot])

        def wwait(slot):
            pltpu.make_async_copy(
                bufS.at[slot], outS.at[pl.ds(0, 512)], wsems[slot]).wait()
            pltpu.make_async_copy(
                bufC.at[slot], outC.at[pl.ds(0, 512)], wsems[slot]).wait()

        fire(0, 0)

        def it(b, slot):
            nxt = 1 - slot

            @pl.when(b >= 1)
            def _():
                wwait(nxt)

            @pl.when(b + 1 < nb)
            def _():
                fire(b + 1, nxt)

            gwait(slot)
            wstart(b, slot)

        def loop_body(b, carry):
            @pl.when(b % 2 == 0)
            def _():
                it(b, 0)

            @pl.when(b % 2 == 1)
            def _():
                it(b, 1)

            return carry

        lax.fori_loop(0, nb, loop_body, 0)
        wwait((nb - 1) % 2)

    f = functools.partial(
        pl.kernel,
        out_type=(jax.ShapeDtypeStruct((Bp, H), jnp.bfloat16),
                  jax.ShapeDtypeStruct((Bp, 16), F32)),
        mesh=_mesh(),
        compiler_params=pltpu.CompilerParams(use_tc_tiling_on_sc=False,
                                             needs_layout_passes=False),
        scratch_types=[
            pltpu.VMEM((kpc, 128), I32),
            pltpu.VMEM((kpc, 128), I32),
            pltpu.VMEM((2, 512, H), jnp.bfloat16),
            pltpu.VMEM((2, 512, 16), F32),
            pltpu.SemaphoreType.DMA,
            pltpu.SemaphoreType.DMA,
            pltpu.SemaphoreType.DMA,
            pltpu.SemaphoreType.DMA,
        ],
    )(body)
    return f(tabA, tabC, srcI, dstI)


def _sc_ffn_add(pf, lab2, zero_acc, Np, l2c16, l2lab2, Bp2):
    cpw = Np // NWORK
    kpc = cpw // 128
    nacc = NACC1
    cpw2 = Bp2 // NWORK
    kpc2 = cpw2 // 128

    def body(pf_ref, lab_ref, zero_ref, ct_ref, ci_ref, out_ref, cg_ref,
             idx_v, rows_v, cix, cbuf, acc):
        w = _wid()
        s = lax.axis_index("s")
        c = lax.axis_index("c")
        stripe = nacc // 16
        pltpu.sync_copy(zero_ref.at[pl.ds(s * stripe, stripe)],
                        acc.at[pl.ds(s * stripe, stripe)])
        plsc.subcore_barrier()
        pltpu.sync_copy(lab_ref.at[pl.ds(w * kpc, kpc)], idx_v)
        pltpu.sync_copy(ci_ref.at[pl.ds(w * kpc2, kpc2)], cix)

        def cg_body(j, carry):
            pltpu.async_copy(ct_ref.at[cix.at[j]], cbuf, None) \
                if False else pltpu.sync_copy(ct_ref.at[cix.at[j]], cbuf)
            pltpu.sync_copy(cbuf,
                            cg_ref.at[pl.ds(w * cpw2 + j * 128, 128)])
            return carry

        lax.fori_loop(0, kpc2, cg_body, 0)

        def loop_body(j, carry):
            pltpu.sync_copy(pf_ref.at[pl.ds(w * cpw + j * 128, 128)], rows_v)
            pltpu.sync_copy(rows_v, acc.at[idx_v.at[j]], add=True)
            return carry

        lax.fori_loop(0, kpc, loop_body, 0)
        plsc.subcore_barrier()

        @pl.when(s == 0)
        def _():
            pltpu.sync_copy(acc.at[pl.ds(0, P1)], out_ref.at[c])

    f = functools.partial(
        pl.kernel,
        out_type=(jax.ShapeDtypeStruct((2, P1, 16), F32),
                  jax.ShapeDtypeStruct((Bp2, 16), F32)),
        mesh=_mesh(),
        compiler_params=pltpu.CompilerParams(use_tc_tiling_on_sc=False,
                                             needs_layout_passes=False),
        scratch_types=[
            pltpu.VMEM((kpc, 128), I32),
            pltpu.VMEM((128, 16), F32),
            pltpu.VMEM((kpc2, 128), I32),
            pltpu.VMEM((128, 16), F32),
            pltpu.VMEM_SHARED((NACC1, 16), F32),
        ],
    )(body)
    return f(pf, lab2, zero_acc, l2c16, l2lab2)


def _sc_smax(zt, dst, n_out, n_acc, Ep):
    nc2 = Ep // 2048
    assert Ep % 2048 == 0

    def body(zt_ref, dst_ref, out_ref, zb, db, a0, a1, a2, a3, f0, f1):
        w = _wid()
        fg = w // 2
        half = w % 2
        accs = (a0, a1, a2, a3)

        def zero_body(i, carry):
            z16 = jnp.zeros((16,), F32)
            for a in accs:
                a[pl.ds(i * 16, 16)] = z16
            return carry

        lax.fori_loop(0, n_acc // 16, zero_body, 0)
        fsems = (f0, f1)
        ebase = half * nc2 * 1024

        def fetch(ci, slot):
            for f in range(4):
                pltpu.make_async_copy(
                    zt_ref.at[pl.ds((4 * fg + f) * Ep + ebase + ci * 1024,
                                    1024)],
                    zb.at[slot, f], fsems[slot]).start()
            pltpu.make_async_copy(
                dst_ref.at[pl.ds(ebase + ci * 1024, 1024)], db.at[slot],
                fsems[slot]).start()

        def fwait(slot):
            for f in range(4):
                pltpu.make_async_copy(
                    zt_ref.at[pl.ds(0, 1024)], zb.at[slot, f],
                    fsems[slot]).wait()
            pltpu.make_async_copy(
                dst_ref.at[pl.ds(0, 1024)], db.at[slot], fsems[slot]).wait()

        def process(slot):
            def group(g, carry):
                dv = db[slot, pl.ds(g * 16, 16)]
                zs = [zb[slot, f, pl.ds(g * 16, 16)] for f in range(4)]
                ns = []
                for f in range(4):
                    o = plsc.load_gather(accs[f], [dv])
                    nv = jnp.maximum(o, zs[f])
                    plsc.store_scatter(accs[f], [dv], nv)
                    ns.append(nv)
                pend = jnp.zeros((16,), I32)
                for f in range(4):
                    c = plsc.load_gather(accs[f], [dv])
                    pend = pend + (c < ns[f]).astype(I32)

                def cond(q):
                    return jnp.max(q) > 0

                def wbody(q):
                    m = q > 0
                    nq = jnp.zeros((16,), I32)
                    for f in range(4):
                        a = plsc.load_gather(accs[f], [dv])
                        u = jnp.maximum(a, zs[f])
                        plsc.store_scatter(accs[f], [dv], u, mask=m)
                        r = plsc.load_gather(accs[f], [dv])
                        nq = nq + jnp.logical_and(m, r < u).astype(I32)
                    return nq

                lax.while_loop(cond, wbody, pend)
                return carry

            lax.fori_loop(0, 64, group, 0)

        fetch(0, 0)

        def it(ci, slot):
            nxt = 1 - slot

            @pl.when(ci + 1 < nc2)
            def _():
                fetch(ci + 1, nxt)

            fwait(slot)
            process(slot)

        def loop_body(ci, carry):
            @pl.when(ci % 2 == 0)
            def _():
                it(ci, 0)

            @pl.when(ci % 2 == 1)
            def _():
                it(ci, 1)

            return carry

        lax.fori_loop(0, nc2, loop_body, 0)
        for f in range(4):
            pltpu.sync_copy(
                accs[f].at[pl.ds(0, n_out)],
                out_ref.at[pl.ds((half * H + 4 * fg + f) * n_out, n_out)])

    f = functools.partial(
        pl.kernel,
        out_type=jax.ShapeDtypeStruct((2 * H * n_out,), F32),
        mesh=_mesh(),
        compiler_params=pltpu.CompilerParams(use_tc_tiling_on_sc=False,
                                             needs_layout_passes=False),
        scratch_types=[
            pltpu.VMEM((2, 4, 1024), F32),
            pltpu.VMEM((2, 1024), I32),
            pltpu.VMEM((n_acc,), F32),
            pltpu.VMEM((n_acc,), F32),
            pltpu.VMEM((n_acc,), F32),
            pltpu.VMEM((n_acc,), F32),
            pltpu.SemaphoreType.DMA,
            pltpu.SemaphoreType.DMA,
        ],
    )(body)
    return f(zt.reshape(-1), dst).reshape(2, H, n_out)


def _spec(block, imap):
    return pl.BlockSpec(block, imap)


def _tc_ffn(part, l1c, W1, b1, W2, b2):
    def body(p_ref, c_ref, w1, bb1, w2, bb2, o_ref):
        s = p_ref[0] + p_ref[1]
        agg = jnp.concatenate(
            [s[:, 0:1], s[:, 1:4] - s[:, 4:5] * c_ref[...]], axis=1)
        t = jnp.maximum(jnp.dot(agg, w1[...]) + bb1[...], 0.0)
        o_ref[...] = jnp.maximum(jnp.dot(t, w2[...]) + bb2[...], 0.0)

    return pl.pallas_call(
        body,
        grid=(P1 // 512,),
        in_specs=[
            _spec((2, 512, 16), lambda i: (0, i, 0)),
            _spec((512, 3), lambda i: (i, 0)),
            _spec((4, H), lambda i: (0, 0)),
            _spec((1, H), lambda i: (0, 0)),
            _spec((H, H), lambda i: (0, 0)),
            _spec((1, H), lambda i: (0, 0)),
        ],
        out_specs=_spec((512, H), lambda i: (i, 0)),
        out_shape=jax.ShapeDtypeStruct((P1, H), F32),
    )(part, l1c, W1, b1, W2, b2)


def _tc_gnnpre(x, c, W1x, W1c, b1, n):
    def body(x_ref, c_ref, wx, wc, bb, a_ref):
        a_ref[...] = (jnp.dot(x_ref[...], wx[...])
                      + jnp.dot(c_ref[...], wc[...])
                      + bb[...]).astype(jnp.bfloat16)

    return pl.pallas_call(
        body,
        grid=(n // 512,),
        in_specs=[
            _spec((512, H), lambda i: (i, 0)),
            _spec((512, 3), lambda i: (i, 0)),
            _spec((H, H), lambda i: (0, 0)),
            _spec((3, H), lambda i: (0, 0)),
            _spec((1, H), lambda i: (0, 0)),
        ],
        out_specs=_spec((512, H), lambda i: (i, 0)),
        out_shape=jax.ShapeDtypeStruct((n, H), jnp.bfloat16),
    )(x, c, W1x, W1c, b1)


def _tc_edge(S, C16, W1c16, W2, b2c, Ep):
    def body(s_ref, c_ref, wc, w2, bb, o_ref):
        s = s_ref[...].astype(F32)
        h = jnp.maximum(s - jnp.dot(c_ref[...], wc[...]), 0.0)
        zt = lax.dot_general(w2[...], h, (((0,), (1,)), ((), ())))
        o_ref[...] = jnp.maximum(zt + bb[...], 0.0)

    return pl.pallas_call(
        body,
        grid=(Ep // 512,),
        in_specs=[
            _spec((512, H), lambda i: (i, 0)),
            _spec((512, 16), lambda i: (i, 0)),
            _spec((16, H), lambda i: (0, 0)),
            _spec((H, H), lambda i: (0, 0)),
            _spec((H, 1), lambda i: (0, 0)),
        ],
        out_specs=_spec((H, 512), lambda i: (0, i)),
        out_shape=jax.ShapeDtypeStruct((H, Ep), F32),
    )(S, C16, W1c16, W2, b2c)


def _tc_post(aggT, x, W4, b4, W5, b5, n, x2=None):
    def body(*refs):
        if x2 is None:
            a_ref, x_ref, w4, bb4, w5, bb5, o_ref = refs
        else:
            a_ref, x_ref, x2_ref, w4, bb4, w5, bb5, o_ref = refs
        a = jnp.maximum(a_ref[0], a_ref[1])
        a4 = lax.dot_general(a, w4[...], (((0,), (0,)), ((), ())))
        y1 = jnp.maximum(a4 + bb4[...], 0.0)
        y2 = jnp.maximum(jnp.dot(y1, w5[...]) + bb5[...], 0.0)
        o = x_ref[...] + y2
        if x2 is not None:
            o = o + x2_ref[...]
        o_ref[...] = o

    in_specs = [_spec((2, H, 512), lambda i: (0, 0, i)),
                _spec((512, H), lambda i: (i, 0))]
    args = [aggT, x]
    if x2 is not None:
        in_specs.append(_spec((512, H), lambda i: (i, 0)))
        args.append(x2)
    in_specs += [
        _spec((H, H), lambda i: (0, 0)),
        _spec((1, H), lambda i: (0, 0)),
        _spec((H, H), lambda i: (0, 0)),
        _spec((1, H), lambda i: (0, 0)),
    ]
    args += [W4, b4, W5, b5]
    return pl.pallas_call(
        body,
        grid=(n // 512,),
        in_specs=in_specs,
        out_specs=_spec((512, H), lambda i: (i, 0)),
        out_shape=jax.ShapeDtypeStruct((n, H), F32),
    )(*args)


def _tc_m2l(t2p, l1cp, cgp, Wa, Wb, bc, Np):
    def body(t_ref, c_ref, g_ref, wa, wb, bb, o_ref):
        m = lax.dot_general(wa[...], t_ref[...], (((0,), (1,)), ((), ())))
        d3 = c_ref[...] - g_ref[...][:, :3]
        m = m + lax.dot_general(wb[...], d3, (((0,), (1,)), ((), ())))
        o_ref[...] = jnp.maximum(m + bb[...], 0.0)

    return pl.pallas_call(
        body,
        grid=(Np // 512,),
        in_specs=[
            _spec((512, H), lambda i: (i, 0)),
            _spec((512, 3), lambda i: (i, 0)),
            _spec((512, 16), lambda i: (i, 0)),
            _spec((H, H), lambda i: (0, 0)),
            _spec((3, H), lambda i: (0, 0)),
            _spec((H, 1), lambda i: (0, 0)),
        ],
        out_specs=_spec((H, 512), lambda i: (0, i)),
        out_shape=jax.ShapeDtypeStruct((H, Np), F32),
    )(t2p, l1cp, cgp, Wa, Wb, bc)


def _tc_trans(xT, n):
    def body(x_ref, o_ref):
        o_ref[...] = jnp.maximum(x_ref[0], x_ref[1]).T

    return pl.pallas_call(
        body,
        grid=(n // 512,),
        in_specs=[_spec((2, H, 512), lambda i: (0, 0, i))],
        out_specs=_spec((512, H), lambda i: (i, 0)),
        out_shape=jax.ShapeDtypeStruct((n, H), F32),
    )(xT)


def _tc_l2mpre(t4, l2c, Wa, Wb, n):
    def body(t_ref, c_ref, wa, wb, o_ref):
        o_ref[...] = jnp.dot(t_ref[...], wa[...]) + jnp.dot(c_ref[...], wb[...])

    return pl.pallas_call(
        body,
        grid=(n // 512,),
        in_specs=[
            _spec((512, H), lambda i: (i, 0)),
            _spec((512, 3), lambda i: (i, 0)),
            _spec((H, H), lambda i: (0, 0)),
            _spec((3, H), lambda i: (0, 0)),
        ],
        out_specs=_spec((512, H), lambda i: (i, 0)),
        out_shape=jax.ShapeDtypeStruct((n, H), F32),
    )(t4, l2c, Wa, Wb)


def _tc_g6pre(gH, l1c, Wbl, bl, W1x, W1c, b1):
    def body(g_ref, c_ref, wbl, bbl, wx, wc, bb1, t5_ref, a_ref):
        t5 = jnp.maximum(
            g_ref[...] - jnp.dot(c_ref[...], wbl[...]) + bbl[...], 0.0)
        t5_ref[...] = t5
        a_ref[...] = (jnp.dot(t5, wx[...]) + jnp.dot(c_ref[...], wc[...])
                      + bb1[...]).astype(jnp.bfloat16)

    return pl.pallas_call(
        body,
        grid=(P1 // 512,),
        in_specs=[
            _spec((512, H), lambda i: (i, 0)),
            _spec((512, 3), lambda i: (i, 0)),
            _spec((3, H), lambda i: (0, 0)),
            _spec((1, H), lambda i: (0, 0)),
            _spec((H, H), lambda i: (0, 0)),
            _spec((3, H), lambda i: (0, 0)),
            _spec((1, H), lambda i: (0, 0)),
        ],
        out_specs=(_spec((512, H), lambda i: (i, 0)),
                   _spec((512, H), lambda i: (i, 0))),
        out_shape=(jax.ShapeDtypeStruct((P1, H), F32),
                   jax.ShapeDtypeStruct((P1, H), jnp.bfloat16)),
    )(gH, l1c, Wbl, bl, W1x, W1c, b1)


def _tc_fbnpre(t6, l1c, Wfa, Wfb, bf):
    def body(t_ref, c_ref, wa, wb, bb, o_ref):
        o_ref[...] = (jnp.dot(t_ref[...], wa[...])
                      - jnp.dot(c_ref[...], wb[...]) + bb[...])

    return pl.pallas_call(
        body,
        grid=(P1 // 512,),
        in_specs=[
            _spec((512, H), lambda i: (i, 0)),
            _spec((512, 3), lambda i: (i, 0)),
            _spec((H, H), lambda i: (0, 0)),
            _spec((3, H), lambda i: (0, 0)),
            _spec((1, H), lambda i: (0, 0)),
        ],
        out_specs=_spec((512, H), lambda i: (i, 0)),
        out_shape=jax.ShapeDtypeStruct((P1, H), F32),
    )(t6, l1c, Wfa, Wfb, bf)


def _tc_final(gG, pts, rem, Wfb, Wfc, Wc, bc):
    def body(g_ref, p_ref, r_ref, wb, wc1, wcls, bcls, o_ref):
        t7 = jnp.maximum(
            g_ref[...] + jnp.dot(p_ref[...], wb[...])
            + jnp.dot(r_ref[...], wc1[...]), 0.0)
        o_ref[...] = jnp.dot(t7, wcls[...]) + bcls[...]

    return pl.pallas_call(
        body,
        grid=(N // 1000,),
        in_specs=[
            _spec((1000, H), lambda i: (i, 0)),
            _spec((1000, 3), lambda i: (i, 0)),
            _spec((1000, 1), lambda i: (i, 0)),
            _spec((3, H), lambda i: (0, 0)),
            _spec((1, H), lambda i: (0, 0)),
            _spec((H, NCLS), lambda i: (0, 0)),
            _spec((1, NCLS), lambda i: (0, 0)),
        ],
        out_specs=_spec((1000, NCLS), lambda i: (i, 0)),
        out_shape=jax.ShapeDtypeStruct((N, NCLS), F32),
    )(gG, pts, rem, Wfb, Wfc, Wc, bc)


def _pad_idx(idx, Bp, fill):
    p = jnp.pad(idx.astype(I32), (0, Bp - idx.shape[0]), constant_values=fill)
    return p


def _gnn_core(A, c16, edges, npad, ntrash, Bp, n_acc, W1c, W2, b2):
    src2 = _pad_idx(edges[0], Bp, 0).reshape(Bp // 128, 128)
    dst2 = _pad_idx(edges[1], Bp, 0).reshape(Bp // 128, 128)
    S, C = _sc_gather_sc(A, c16, src2, dst2, npad, Bp)
    W1c16 = jnp.pad(W1c, ((0, 13), (0, 0)))
    zt = _tc_edge(S, C, W1c16, W2, b2.reshape(H, 1), Bp)
    dsts = _pad_idx(edges[1], Bp, ntrash)
    return _sc_smax(zt, dsts, npad, n_acc, Bp)


def _gnn_layer(x, centers, c16, edges, p, npad, ntrash, Bp, n_acc, x2=None):
    W1, b1, W2, b2, W4, b4, W5, b5 = p
    W1x, W1c = W1[:H], W1[H:]
    A = _tc_gnnpre(x, centers, W1x, W1c, b1.reshape(1, H), npad)
    aggT = _gnn_core(A, c16, edges, npad, ntrash, Bp, n_acc, W1c, W2, b2)
    return _tc_post(aggT, x, W4, b4.reshape(1, H), W5, b5.reshape(1, H), npad,
                    x2=x2)


def kernel(remission, points, l1_cluster_centers, l2_cluster_centers,
           l1_edges, l2_edges, l1_labels, l2_labels, params):
    l1c = jnp.pad(l1_cluster_centers, ((0, P1 - N1), (0, 0)))
    l2c = jnp.pad(l2_cluster_centers, ((0, P2 - N2), (0, 0)))
    l1_labels = l1_labels.astype(I32)
    l2_labels = l2_labels.astype(I32)
    l1_edges = l1_edges.astype(I32)
    l2_edges = l2_edges.astype(I32)

    NP_FFN = 131072
    BP1 = 425984
    BP2 = 98304
    BPL1 = 131072
    BPL2 = 32768
    NP_M2L = 26624

    pf = jnp.concatenate(
        [remission, points, jnp.ones((N, 1), F32), jnp.zeros((N, 11), F32)],
        axis=1)
    pf = jnp.pad(pf, ((0, NP_FFN - N), (0, 0)))
    lab2 = _pad_idx(l1_labels, NP_FFN, N1).reshape(NP_FFN // 128, 128)
    zero_acc = jnp.zeros((NACC1, 16), F32)
    l2c16 = jnp.pad(l2_cluster_centers, ((0, P2 - N2), (0, 13)))
    l2lab2 = _pad_idx(l2_labels, BPL2, 0).reshape(BPL2 // 128, 128)
    part, cg = _sc_ffn_add(pf, lab2, zero_acc, NP_FFN, l2c16, l2lab2, BPL2)
    fW1, fb1, fW2, fb2 = params["ffn"]
    t1 = _tc_ffn(part, l1c, fW1, fb1.reshape(1, H), fW2, fb2.reshape(1, H))

    l1c16 = jnp.pad(l1c, ((0, 0), (0, 13)))
    t2 = _gnn_layer(t1, l1c, l1c16, l1_edges, params["g2"], P1, N1, BP1,
                    NACC1)

    mW, mb = params["m2l"]
    t2p = jnp.pad(t2, ((0, NP_M2L - P1), (0, 0)))
    l1cp = jnp.pad(l1c, ((0, NP_M2L - P1), (0, 0)))
    cgp = cg[:NP_M2L]
    mfT = _tc_m2l(t2p, l1cp, cgp, mW[:H], mW[H:], mb.reshape(H, 1), NP_M2L)
    mdst = _pad_idx(l2_labels, NP_M2L, N2)
    t3T = _sc_smax(mfT, mdst, P2, NACC2, NP_M2L)
    t3 = _tc_trans(t3T, P2)

    t4 = _gnn_layer(t3, l2c, l2c16, l2_edges, params["g4"], P2, N2, BP2,
                    NACC2)
    t4 = _gnn_layer(t4, l2c, l2c16, l2_edges, params["g41"], P2, N2, BP2,
                    NACC2)

    lW, lb = params["l2m"]
    H2 = _tc_l2mpre(t4, l2c, lW[:H], lW[H:], P2)
    gH = _sc_gather(H2, _pad_idx(l2_labels, BPL2, 0).reshape(-1, 128),
                    P2, BPL2)[:P1]
    g6W1, g6b1 = params["g6"][0], params["g6"][1]
    t5, A6 = _tc_g6pre(gH, l1c, lW[H:], lb.reshape(1, H),
                       g6W1[:H], g6W1[H:], g6b1.reshape(1, H))
    agg6T = _gnn_core(A6, l1c16, l1_edges, P1, N1, BP1, NACC1,
                      g6W1[H:], params["g6"][2], params["g6"][3])
    t6 = _tc_post(agg6T, t5, params["g6"][4], params["g6"][5].reshape(1, H),
                  params["g6"][6], params["g6"][7].reshape(1, H), P1, x2=t2)

    fbW, fbb = params["fbn"]
    G = _tc_fbnpre(t6, l1c, fbW[:H], fbW[H:H + 3], fbb.reshape(1, H))
    gG = _sc_gather(G, _pad_idx(l1_labels, BPL1, 0).reshape(-1, 128),
                    P1, BPL1)[:N]
    cW, cb = params["cls"]
    return _tc_final(gG, points, remission, fbW[H:H + 3], fbW[H + 3:],
                     cW, cb.reshape(1, NCLS))


# speedup vs baseline: 1.4775x; 1.0055x over previous
import functools

import jax
import jax.numpy as jnp
from jax import lax
from jax.experimental import pallas as pl
from jax.experimental.pallas import tpu as pltpu
from jax.experimental.pallas import tpu_sc as plsc

F32 = jnp.float32
I32 = jnp.int32
N, N1, N2 = 100000, 25000, 5000
E1, E2 = 400000, 80000
H = 64
NCLS = 20
NWORK = 32
P1 = 25600
P2 = 5120
NACC1 = P1 + 128
NACC2 = P2 + 128


def _mesh():
    return plsc.VectorSubcoreMesh(core_axis_name="c", subcore_axis_name="s")


def _wid():
    return lax.axis_index("s") * 2 + lax.axis_index("c")


def _sc_gather(table, idx2, V, Bp, D=H):
    cpw = Bp // NWORK
    nb = cpw // 512
    kpc = cpw // 128
    assert cpw % 512 == 0

    def body(tab_ref, idx_ref, out_ref, idx_v, bufs, g0, g1, w0, w1):
        w = _wid()
        pltpu.sync_copy(idx_ref.at[pl.ds(w * kpc, kpc)], idx_v)
        gsems = (g0, g1)
        wsems = (w0, w1)

        def fire(b, slot):
            for j in range(4):
                pltpu.make_async_copy(
                    tab_ref.at[idx_v.at[b * 4 + j]],
                    bufs.at[slot, pl.ds(j * 128, 128)],
                    gsems[slot]).start()

        def gwait(slot):
            pltpu.make_async_copy(
                tab_ref.at[pl.ds(0, 512)], bufs.at[slot], gsems[slot]).wait()

        def wstart(b, slot):
            pltpu.async_copy(
                bufs.at[slot], out_ref.at[pl.ds(w * cpw + b * 512, 512)],
                wsems[slot])

        def wwait(slot):
            pltpu.make_async_copy(
                bufs.at[slot], out_ref.at[pl.ds(0, 512)], wsems[slot]).wait()

        fire(0, 0)

        def it(b, slot):
            nxt = 1 - slot

            @pl.when(b >= 1)
            def _():
                wwait(nxt)

            @pl.when(b + 1 < nb)
            def _():
                fire(b + 1, nxt)

            gwait(slot)
            wstart(b, slot)

        def loop_body(b, carry):
            @pl.when(b % 2 == 0)
            def _():
                it(b, 0)

            @pl.when(b % 2 == 1)
            def _():
                it(b, 1)

            return carry

        lax.fori_loop(0, nb, loop_body, 0)
        wwait((nb - 1) % 2)

    f = functools.partial(
        pl.kernel,
        out_type=jax.ShapeDtypeStruct((Bp, D), F32),
        mesh=_mesh(),
        compiler_params=pltpu.CompilerParams(use_tc_tiling_on_sc=False,
                                             needs_layout_passes=False),
        scratch_types=[
            pltpu.VMEM((kpc, 128), I32),
            pltpu.VMEM((2, 512, D), F32),
            pltpu.SemaphoreType.DMA,
            pltpu.SemaphoreType.DMA,
            pltpu.SemaphoreType.DMA,
            pltpu.SemaphoreType.DMA,
        ],
    )(body)
    return f(table, idx2)


def _sc_gather_sc(tabA, tabC, srcI, dstI, V, Bp):
    cpw = Bp // NWORK
    nb = cpw // 512
    kpc = cpw // 128
    assert cpw % 512 == 0

    def body(ta_ref, tc_ref, si_ref, di_ref, outS, outC,
             six, dix, bufS, bufC, g0, g1, w0, w1):
        w = _wid()
        pltpu.sync_copy(si_ref.at[pl.ds(w * kpc, kpc)], six)
        pltpu.sync_copy(di_ref.at[pl.ds(w * kpc, kpc)], dix)
        gsems = (g0, g1)
        wsems = (w0, w1)

        def fire(b, slot):
            for j in range(4):
                pltpu.make_async_copy(
                    ta_ref.at[six.at[b * 4 + j]],
                    bufS.at[slot, pl.ds(j * 128, 128)],
                    gsems[slot]).start()
                pltpu.make_async_copy(
                    tc_ref.at[dix.at[b * 4 + j]],
                    bufC.at[slot, pl.ds(j * 128, 128)],
                    gsems[slot]).start()

        def gwait(slot):
            pltpu.make_async_copy(
                ta_ref.at[pl.ds(0, 512)], bufS.at[slot], gsems[slot]).wait()
            pltpu.make_async_copy(
                tc_ref.at[pl.ds(0, 512)], bufC.at[slot], gsems[slot]).wait()

        def wstart(b, slot):
            pltpu.async_copy(
                bufS.at[slot], outS.at[pl.ds(w * cpw + b * 512, 512)],
                wsems[slot])
            pltpu.async_copy(
                bufC.at[slot], outC.at[pl.ds(w * cpw + b * 512, 512)],
                wsems[slot])

        def wwait(slot):
            pltpu.make_async_copy(
                bufS.at[slot], outS.at[pl.ds(0, 512)], wsems[slot]).wait()
            pltpu.make_async_copy(
                bufC.at[slot], outC.at[pl.ds(0, 512)], wsems[slot]).wait()

        fire(0, 0)

        def it(b, slot):
            nxt = 1 - slot

            @pl.when(b >= 1)
            def _():
                wwait(nxt)

            @pl.when(b + 1 < nb)
            def _():
                fire(b + 1, nxt)

            gwait(slot)
            wstart(b, slot)

        def loop_body(b, carry):
            @pl.when(b % 2 == 0)
            def _():
                it(b, 0)

            @pl.when(b % 2 == 1)
            def _():
                it(b, 1)

            return carry

        lax.fori_loop(0, nb, loop_body, 0)
        wwait((nb - 1) % 2)

    f = functools.partial(
        pl.kernel,
        out_type=(jax.ShapeDtypeStruct((Bp, H), jnp.bfloat16),
                  jax.ShapeDtypeStruct((Bp, 16), F32)),
        mesh=_mesh(),
        compiler_params=pltpu.CompilerParams(use_tc_tiling_on_sc=False,
                                             needs_layout_passes=False),
        scratch_types=[
            pltpu.VMEM((kpc, 128), I32),
            pltpu.VMEM((kpc, 128), I32),
            pltpu.VMEM((2, 512, H), jnp.bfloat16),
            pltpu.VMEM((2, 512, 16), F32),
            pltpu.SemaphoreType.DMA,
            pltpu.SemaphoreType.DMA,
            pltpu.SemaphoreType.DMA,
            pltpu.SemaphoreType.DMA,
        ],
    )(body)
    return f(tabA, tabC, srcI, dstI)


def _sc_ffn_add(pf, lab2, zero_acc, Np, l2c16, l2lab2, Bp2):
    cpw = Np // NWORK
    kpc = cpw // 128
    nacc = NACC1
    cpw2 = Bp2 // NWORK
    kpc2 = cpw2 // 128

    def body(pf_ref, lab_ref, zero_ref, ct_ref, ci_ref, out_ref, cg_ref,
             idx_v, rows_v, cix, cbuf, acc):
        w = _wid()
        s = lax.axis_index("s")
        c = lax.axis_index("c")
        stripe = nacc // 16
        pltpu.sync_copy(zero_ref.at[pl.ds(s * stripe, stripe)],
                        acc.at[pl.ds(s * stripe, stripe)])
        plsc.subcore_barrier()
        pltpu.sync_copy(lab_ref.at[pl.ds(w * kpc, kpc)], idx_v)
        pltpu.sync_copy(ci_ref.at[pl.ds(w * kpc2, kpc2)], cix)

        def cg_body(j, carry):
            pltpu.async_copy(ct_ref.at[cix.at[j]], cbuf, None) \
                if False else pltpu.sync_copy(ct_ref.at[cix.at[j]], cbuf)
            pltpu.sync_copy(cbuf,
                            cg_ref.at[pl.ds(w * cpw2 + j * 128, 128)])
            return carry

        lax.fori_loop(0, kpc2, cg_body, 0)

        def loop_body(j, carry):
            pltpu.sync_copy(pf_ref.at[pl.ds(w * cpw + j * 128, 128)], rows_v)
            pltpu.sync_copy(rows_v, acc.at[idx_v.at[j]], add=True)
            return carry

        lax.fori_loop(0, kpc, loop_body, 0)
        plsc.subcore_barrier()

        @pl.when(s == 0)
        def _():
            pltpu.sync_copy(acc.at[pl.ds(0, P1)], out_ref.at[c])

    f = functools.partial(
        pl.kernel,
        out_type=(jax.ShapeDtypeStruct((2, P1, 16), F32),
                  jax.ShapeDtypeStruct((Bp2, 16), F32)),
        mesh=_mesh(),
        compiler_params=pltpu.CompilerParams(use_tc_tiling_on_sc=False,
                                             needs_layout_passes=False),
        scratch_types=[
            pltpu.VMEM((kpc, 128), I32),
            pltpu.VMEM((128, 16), F32),
            pltpu.VMEM((kpc2, 128), I32),
            pltpu.VMEM((128, 16), F32),
            pltpu.VMEM_SHARED((NACC1, 16), F32),
        ],
    )(body)
    return f(pf, lab2, zero_acc, l2c16, l2lab2)


def _sc_smax(zt, dst, n_out, n_acc, Ep):
    nc2 = Ep // 2048
    assert Ep % 2048 == 0

    def body(zt_ref, dst_ref, out_ref, zb, db, a0, a1, a2, a3, f0, f1):
        w = _wid()
        fg = w // 2
        half = w % 2
        accs = (a0, a1, a2, a3)

        def zero_body(i, carry):
            z16 = jnp.zeros((16,), F32)
            for a in accs:
                a[pl.ds(i * 16, 16)] = z16
            return carry

        lax.fori_loop(0, n_acc // 16, zero_body, 0)
        fsems = (f0, f1)
        ebase = half * nc2 * 1024

        def fetch(ci, slot):
            for f in range(4):
                pltpu.make_async_copy(
                    zt_ref.at[pl.ds((4 * fg + f) * Ep + ebase + ci * 1024,
                                    1024)],
                    zb.at[slot, f], fsems[slot]).start()
            pltpu.make_async_copy(
                dst_ref.at[pl.ds(ebase + ci * 1024, 1024)], db.at[slot],
                fsems[slot]).start()

        def fwait(slot):
            for f in range(4):
                pltpu.make_async_copy(
                    zt_ref.at[pl.ds(0, 1024)], zb.at[slot, f],
                    fsems[slot]).wait()
            pltpu.make_async_copy(
                dst_ref.at[pl.ds(0, 1024)], db.at[slot], fsems[slot]).wait()

        def process(slot):
            def group(g, carry):
                dv = db[slot, pl.ds(g * 16, 16)]
                zs = [zb[slot, f, pl.ds(g * 16, 16)] for f in range(4)]
                ns = []
                for f in range(4):
                    o = plsc.load_gather(accs[f], [dv])
                    nv = jnp.maximum(o, zs[f])
                    plsc.store_scatter(accs[f], [dv], nv)
                    ns.append(nv)
                pend = jnp.zeros((16,), I32)
                for f in range(4):
                    c = plsc.load_gather(accs[f], [dv])
                    pend = pend + (c < ns[f]).astype(I32)

                def cond(q):
                    return jnp.max(q) > 0

                def wbody(q):
                    m = q > 0
                    nq = jnp.zeros((16,), I32)
                    for f in range(4):
                        a = plsc.load_gather(accs[f], [dv])
                        u = jnp.maximum(a, zs[f])
                        plsc.store_scatter(accs[f], [dv], u, mask=m)
                        r = plsc.load_gather(accs[f], [dv])
                        nq = nq + jnp.logical_and(m, r < u).astype(I32)
                    return nq

                lax.while_loop(cond, wbody, pend)
                return carry

            lax.fori_loop(0, 64, group, 0)

        fetch(0, 0)

        def it(ci, slot):
            nxt = 1 - slot

            @pl.when(ci + 1 < nc2)
            def _():
                fetch(ci + 1, nxt)

            fwait(slot)
            process(slot)

        def loop_body(ci, carry):
            @pl.when(ci % 2 == 0)
            def _():
                it(ci, 0)

            @pl.when(ci % 2 == 1)
            def _():
                it(ci, 1)

            return carry

        lax.fori_loop(0, nc2, loop_body, 0)
        for f in range(4):
            pltpu.sync_copy(
                accs[f].at[pl.ds(0, n_out)],
                out_ref.at[pl.ds((half * H + 4 * fg + f) * n_out, n_out)])

    f = functools.partial(
        pl.kernel,
        out_type=jax.ShapeDtypeStruct((2 * H * n_out,), F32),
        mesh=_mesh(),
        compiler_params=pltpu.CompilerParams(use_tc_tiling_on_sc=False,
                                             needs_layout_passes=False),
        scratch_types=[
            pltpu.VMEM((2, 4, 1024), F32),
            pltpu.VMEM((2, 1024), I32),
            pltpu.VMEM((n_acc,), F32),
            pltpu.VMEM((n_acc,), F32),
            pltpu.VMEM((n_acc,), F32),
            pltpu.VMEM((n_acc,), F32),
            pltpu.SemaphoreType.DMA,
            pltpu.SemaphoreType.DMA,
        ],
    )(body)
    return f(zt.reshape(-1), dst).reshape(2, H, n_out)


def _spec(block, imap):
    return pl.BlockSpec(block, imap)


def _tc_ffn(part, l1c, W1, b1, W2, b2):
    def body(p_ref, c_ref, w1, bb1, w2, bb2, o_ref):
        s = p_ref[0] + p_ref[1]
        agg = jnp.concatenate(
            [s[:, 0:1], s[:, 1:4] - s[:, 4:5] * c_ref[...]], axis=1)
        t = jnp.maximum(jnp.dot(agg, w1[...]) + bb1[...], 0.0)
        o_ref[...] = jnp.maximum(jnp.dot(t, w2[...]) + bb2[...], 0.0)

    return pl.pallas_call(
        body,
        grid=(P1 // 512,),
        in_specs=[
            _spec((2, 512, 16), lambda i: (0, i, 0)),
            _spec((512, 3), lambda i: (i, 0)),
            _spec((4, H), lambda i: (0, 0)),
            _spec((1, H), lambda i: (0, 0)),
            _spec((H, H), lambda i: (0, 0)),
            _spec((1, H), lambda i: (0, 0)),
        ],
        out_specs=_spec((512, H), lambda i: (i, 0)),
        out_shape=jax.ShapeDtypeStruct((P1, H), F32),
    )(part, l1c, W1, b1, W2, b2)


def _tc_gnnpre(x, c, W1x, W1c, b1, n):
    def body(x_ref, c_ref, wx, wc, bb, a_ref):
        a_ref[...] = (jnp.dot(x_ref[...], wx[...])
                      + jnp.dot(c_ref[...], wc[...])
                      + bb[...]).astype(jnp.bfloat16)

    return pl.pallas_call(
        body,
        grid=(n // 512,),
        in_specs=[
            _spec((512, H), lambda i: (i, 0)),
            _spec((512, 3), lambda i: (i, 0)),
            _spec((H, H), lambda i: (0, 0)),
            _spec((3, H), lambda i: (0, 0)),
            _spec((1, H), lambda i: (0, 0)),
        ],
        out_specs=_spec((512, H), lambda i: (i, 0)),
        out_shape=jax.ShapeDtypeStruct((n, H), jnp.bfloat16),
    )(x, c, W1x, W1c, b1)


def _tc_edge(S, C16, W1c16, W2, b2c, Ep):
    def body(s_ref, c_ref, wc, w2, bb, o_ref):
        s = s_ref[...].astype(F32)
        h = jnp.maximum(s - jnp.dot(c_ref[...], wc[...]), 0.0)
        zt = lax.dot_general(w2[...], h, (((0,), (1,)), ((), ())))
        o_ref[...] = jnp.maximum(zt + bb[...], 0.0)

    return pl.pallas_call(
        body,
        grid=(Ep // 512,),
        in_specs=[
            _spec((512, H), lambda i: (i, 0)),
            _spec((512, 16), lambda i: (i, 0)),
            _spec((16, H), lambda i: (0, 0)),
            _spec((H, H), lambda i: (0, 0)),
            _spec((H, 1), lambda i: (0, 0)),
        ],
        out_specs=_spec((H, 512), lambda i: (0, i)),
        out_shape=jax.ShapeDtypeStruct((H, Ep), F32),
    )(S, C16, W1c16, W2, b2c)


def _tc_post(aggT, x, W4, b4, W5, b5, n, x2=None):
    def body(*refs):
        if x2 is None:
            a_ref, x_ref, w4, bb4, w5, bb5, o_ref = refs
        else:
            a_ref, x_ref, x2_ref, w4, bb4, w5, bb5, o_ref = refs
        a = jnp.maximum(a_ref[0], a_ref[1])
        a4 = lax.dot_general(a, w4[...], (((0,), (0,)), ((), ())))
        y1 = jnp.maximum(a4 + bb4[...], 0.0)
        y2 = jnp.maximum(jnp.dot(y1, w5[...]) + bb5[...], 0.0)
        o = x_ref[...] + y2
        if x2 is not None:
            o = o + x2_ref[...]
        o_ref[...] = o

    in_specs = [_spec((2, H, 512), lambda i: (0, 0, i)),
                _spec((512, H), lambda i: (i, 0))]
    args = [aggT, x]
    if x2 is not None:
        in_specs.append(_spec((512, H), lambda i: (i, 0)))
        args.append(x2)
    in_specs += [
        _spec((H, H), lambda i: (0, 0)),
        _spec((1, H), lambda i: (0, 0)),
        _spec((H, H), lambda i: (0, 0)),
        _spec((1, H), lambda i: (0, 0)),
    ]
    args += [W4, b4, W5, b5]
    return pl.pallas_call(
        body,
        grid=(n // 512,),
        in_specs=in_specs,
        out_specs=_spec((512, H), lambda i: (i, 0)),
        out_shape=jax.ShapeDtypeStruct((n, H), F32),
    )(*args)


def _tc_m2l(t2p, l1cp, cgp, Wa, Wb, bc, Np):
    def body(t_ref, c_ref, g_ref, wa, wb, bb, o_ref):
        m = lax.dot_general(wa[...], t_ref[...], (((0,), (1,)), ((), ())))
        d3 = c_ref[...] - g_ref[...][:, :3]
        m = m + lax.dot_general(wb[...], d3, (((0,), (1,)), ((), ())))
        o_ref[...] = jnp.maximum(m + bb[...], 0.0)

    return pl.pallas_call(
        body,
        grid=(Np // 512,),
        in_specs=[
            _spec((512, H), lambda i: (i, 0)),
            _spec((512, 3), lambda i: (i, 0)),
            _spec((512, 16), lambda i: (i, 0)),
            _spec((H, H), lambda i: (0, 0)),
            _spec((3, H), lambda i: (0, 0)),
            _spec((H, 1), lambda i: (0, 0)),
        ],
        out_specs=_spec((H, 512), lambda i: (0, i)),
        out_shape=jax.ShapeDtypeStruct((H, Np), F32),
    )(t2p, l1cp, cgp, Wa, Wb, bc)


def _tc_trans(xT, n):
    def body(x_ref, o_ref):
        o_ref[...] = jnp.maximum(x_ref[0], x_ref[1]).T

    return pl.pallas_call(
        body,
        grid=(n // 512,),
        in_specs=[_spec((2, H, 512), lambda i: (0, 0, i))],
        out_specs=_spec((512, H), lambda i: (i, 0)),
        out_shape=jax.ShapeDtypeStruct((n, H), F32),
    )(xT)


def _tc_l2mpre(t4, l2c, Wa, Wb, n):
    def body(t_ref, c_ref, wa, wb, o_ref):
        o_ref[...] = jnp.dot(t_ref[...], wa[...]) + jnp.dot(c_ref[...], wb[...])

    return pl.pallas_call(
        body,
        grid=(n // 512,),
        in_specs=[
            _spec((512, H), lambda i: (i, 0)),
            _spec((512, 3), lambda i: (i, 0)),
            _spec((H, H), lambda i: (0, 0)),
            _spec((3, H), lambda i: (0, 0)),
        ],
        out_specs=_spec((512, H), lambda i: (i, 0)),
        out_shape=jax.ShapeDtypeStruct((n, H), F32),
    )(t4, l2c, Wa, Wb)


def _tc_g6pre(gH, l1c, Wbl, bl, W1x, W1c, b1):
    def body(g_ref, c_ref, wbl, bbl, wx, wc, bb1, t5_ref, a_ref):
        t5 = jnp.maximum(
            g_ref[...] - jnp.dot(c_ref[...], wbl[...]) + bbl[...], 0.0)
        t5_ref[...] = t5
        a_ref[...] = (jnp.dot(t5, wx[...]) + jnp.dot(c_ref[...], wc[...])
                      + bb1[...]).astype(jnp.bfloat16)

    return pl.pallas_call(
        body,
        grid=(P1 // 512,),
        in_specs=[
            _spec((512, H), lambda i: (i, 0)),
            _spec((512, 3), lambda i: (i, 0)),
            _spec((3, H), lambda i: (0, 0)),
            _spec((1, H), lambda i: (0, 0)),
            _spec((H, H), lambda i: (0, 0)),
            _spec((3, H), lambda i: (0, 0)),
            _spec((1, H), lambda i: (0, 0)),
        ],
        out_specs=(_spec((512, H), lambda i: (i, 0)),
                   _spec((512, H), lambda i: (i, 0))),
        out_shape=(jax.ShapeDtypeStruct((P1, H), F32),
                   jax.ShapeDtypeStruct((P1, H), jnp.bfloat16)),
    )(gH, l1c, Wbl, bl, W1x, W1c, b1)


def _tc_fbnpre(t6, l1c, Wfa, Wfb, bf):
    def body(t_ref, c_ref, wa, wb, bb, o_ref):
        o_ref[...] = (jnp.dot(t_ref[...], wa[...])
                      - jnp.dot(c_ref[...], wb[...]) + bb[...])

    return pl.pallas_call(
        body,
        grid=(P1 // 512,),
        in_specs=[
            _spec((512, H), lambda i: (i, 0)),
            _spec((512, 3), lambda i: (i, 0)),
            _spec((H, H), lambda i: (0, 0)),
            _spec((3, H), lambda i: (0, 0)),
            _spec((1, H), lambda i: (0, 0)),
        ],
        out_specs=_spec((512, H), lambda i: (i, 0)),
        out_shape=jax.ShapeDtypeStruct((P1, H), F32),
    )(t6, l1c, Wfa, Wfb, bf)


def _tc_final(gG, pts, rem, Wfb, Wfc, Wc, bc):
    def body(g_ref, p_ref, r_ref, wb, wc1, wcls, bcls, o_ref):
        t7 = jnp.maximum(
            g_ref[...] + jnp.dot(p_ref[...], wb[...])
            + jnp.dot(r_ref[...], wc1[...]), 0.0)
        o_ref[...] = jnp.dot(t7, wcls[...]) + bcls[...]

    return pl.pallas_call(
        body,
        grid=(N // 1000,),
        in_specs=[
            _spec((1000, H), lambda i: (i, 0)),
            _spec((1000, 3), lambda i: (i, 0)),
            _spec((1000, 1), lambda i: (i, 0)),
            _spec((3, H), lambda i: (0, 0)),
            _spec((1, H), lambda i: (0, 0)),
            _spec((H, NCLS), lambda i: (0, 0)),
            _spec((1, NCLS), lambda i: (0, 0)),
        ],
        out_specs=_spec((1000, NCLS), lambda i: (i, 0)),
        out_shape=jax.ShapeDtypeStruct((N, NCLS), F32),
    )(gG, pts, rem, Wfb, Wfc, Wc, bc)


def _pad_idx(idx, Bp, fill):
    p = jnp.pad(idx.astype(I32), (0, Bp - idx.shape[0]), constant_values=fill)
    return p


def _prep_edges(edges, npad, ntrash, Bp):
    src2 = _pad_idx(edges[0], Bp, 0).reshape(Bp // 128, 128)
    dst2 = _pad_idx(edges[1], Bp, 0).reshape(Bp // 128, 128)
    dsts = _pad_idx(edges[1], Bp, ntrash)
    return (src2, dst2, dsts)


def _gnn_core(A, c16, eprep, npad, Bp, n_acc, W1c, W2, b2):
    src2, dst2, dsts = eprep
    S, C = _sc_gather_sc(A, c16, src2, dst2, npad, Bp)
    W1c16 = jnp.pad(W1c, ((0, 13), (0, 0)))
    zt = _tc_edge(S, C, W1c16, W2, b2.reshape(H, 1), Bp)
    return _sc_smax(zt, dsts, npad, n_acc, Bp)


def _gnn_layer(x, centers, c16, eprep, p, npad, Bp, n_acc, x2=None):
    W1, b1, W2, b2, W4, b4, W5, b5 = p
    W1x, W1c = W1[:H], W1[H:]
    A = _tc_gnnpre(x, centers, W1x, W1c, b1.reshape(1, H), npad)
    aggT = _gnn_core(A, c16, eprep, npad, Bp, n_acc, W1c, W2, b2)
    return _tc_post(aggT, x, W4, b4.reshape(1, H), W5, b5.reshape(1, H), npad,
                    x2=x2)


def kernel(remission, points, l1_cluster_centers, l2_cluster_centers,
           l1_edges, l2_edges, l1_labels, l2_labels, params):
    l1c = jnp.pad(l1_cluster_centers, ((0, P1 - N1), (0, 0)))
    l2c = jnp.pad(l2_cluster_centers, ((0, P2 - N2), (0, 0)))
    l1_labels = l1_labels.astype(I32)
    l2_labels = l2_labels.astype(I32)
    l1_edges = l1_edges.astype(I32)
    l2_edges = l2_edges.astype(I32)

    NP_FFN = 131072
    BP1 = 425984
    BP2 = 98304
    BPL1 = 131072
    BPL2 = 32768
    NP_M2L = 26624

    pf = jnp.concatenate(
        [remission, points, jnp.ones((N, 1), F32), jnp.zeros((N, 11), F32)],
        axis=1)
    pf = jnp.pad(pf, ((0, NP_FFN - N), (0, 0)))
    lab2 = _pad_idx(l1_labels, NP_FFN, N1).reshape(NP_FFN // 128, 128)
    zero_acc = jnp.zeros((NACC1, 16), F32)
    l2c16 = jnp.pad(l2_cluster_centers, ((0, P2 - N2), (0, 13)))
    l2lab2 = _pad_idx(l2_labels, BPL2, 0).reshape(BPL2 // 128, 128)
    part, cg = _sc_ffn_add(pf, lab2, zero_acc, NP_FFN, l2c16, l2lab2, BPL2)
    fW1, fb1, fW2, fb2 = params["ffn"]
    t1 = _tc_ffn(part, l1c, fW1, fb1.reshape(1, H), fW2, fb2.reshape(1, H))

    l1c16 = jnp.pad(l1c, ((0, 0), (0, 13)))
    e1p = _prep_edges(l1_edges, P1, N1, BP1)
    e2p = _prep_edges(l2_edges, P2, N2, BP2)
    t2 = _gnn_layer(t1, l1c, l1c16, e1p, params["g2"], P1, BP1, NACC1)

    mW, mb = params["m2l"]
    t2p = jnp.pad(t2, ((0, NP_M2L - P1), (0, 0)))
    l1cp = jnp.pad(l1c, ((0, NP_M2L - P1), (0, 0)))
    mfT = _tc_m2l(t2p, l1cp, cg, mW[:H], mW[H:], mb.reshape(H, 1), NP_M2L)
    mdst = _pad_idx(l2_labels, NP_M2L, N2)
    t3T = _sc_smax(mfT, mdst, P2, NACC2, NP_M2L)
    t3 = _tc_trans(t3T, P2)

    t4 = _gnn_layer(t3, l2c, l2c16, e2p, params["g4"], P2, BP2, NACC2)
    t4 = _gnn_layer(t4, l2c, l2c16, e2p, params["g41"], P2, BP2, NACC2)

    lW, lb = params["l2m"]
    H2 = _tc_l2mpre(t4, l2c, lW[:H], lW[H:], P2)
    gH = _sc_gather(H2, _pad_idx(l2_labels, BPL2, 0).reshape(-1, 128),
                    P2, BPL2)
    g6W1, g6b1 = params["g6"][0], params["g6"][1]
    t5, A6 = _tc_g6pre(gH, l1c, lW[H:], lb.reshape(1, H),
                       g6W1[:H], g6W1[H:], g6b1.reshape(1, H))
    agg6T = _gnn_core(A6, l1c16, e1p, P1, BP1, NACC1,
                      g6W1[H:], params["g6"][2], params["g6"][3])
    t6 = _tc_post(agg6T, t5, params["g6"][4], params["g6"][5].reshape(1, H),
                  params["g6"][6], params["g6"][7].reshape(1, H), P1, x2=t2)

    fbW, fbb = params["fbn"]
    G = _tc_fbnpre(t6, l1c, fbW[:H], fbW[H:H + 3], fbb.reshape(1, H))
    gG = _sc_gather(G, _pad_idx(l1_labels, BPL1, 0).reshape(-1, 128),
                    P1, BPL1)
    cW, cb = params["cls"]
    return _tc_final(gG, points, remission, fbW[H:H + 3], fbW[H + 3:],
                     cW, cb.reshape(1, NCLS))
